# Initial kernel scaffold; baseline (speedup 1.0000x reference)
import functools

import jax
import jax.numpy as jnp
from jax import lax
from jax.experimental import pallas as pl
from jax.experimental.pallas import tpu as pltpu
from jax.experimental.pallas import tpu_sc as plsc

N, E, D, T, R, C, H, OUT = 10000, 320000, 128, 3, 4, 32, 256, 32

NB = 10
BN = N // NB
NW = 32
CHUNK = 128
E_PAD = 323584
CPW = E_PAD // (NW * CHUNK)
N_PAD = 10016
RPS = N_PAD // 16

_P = jax.lax.Precision.HIGHEST


def _dot(a, b):
    return jnp.dot(a, b, precision=_P, preferred_element_type=jnp.float32)


def _leaky(v):
    return jnp.where(v >= 0, v, 0.01 * v)


def _norm_coefs(st, g, be):
    cnt = jnp.maximum(st[2], 1.0)
    mean = st[0] / cnt
    var = jnp.maximum(st[1] / cnt - mean * mean, 0.0)
    sc = jax.lax.rsqrt(var + 1e-5) * g
    return sc, be - mean * sc


def _accum_stats(so_ref, m, y):
    @pl.when(pl.program_id(0) == 0)
    def _():
        so_ref[...] = jnp.zeros_like(so_ref)

    mt = m.T
    so_ref[0] += _dot(mt, y)
    so_ref[1] += _dot(mt, y * y)
    so_ref[2] += _dot(mt, jnp.ones_like(y))


def _stage_body(norm, res, x_ref, m_ref, w_ref, b_ref, *rest):
    i = 0
    if norm:
        st_ref, g_ref, be_ref = rest[i:i + 3]
        i += 3
    if res:
        r_ref = rest[i]
        i += 1
    y_ref, so_ref = rest[i:i + 2]

    x = x_ref[...]
    m = m_ref[...]
    if norm:
        sc, sh = _norm_coefs(st_ref[...], g_ref[...], be_ref[...])
        x = _leaky(x * _dot(m, sc) + _dot(m, sh))
    y = _dot(m, b_ref[...])
    for t in range(T):
        y += _dot(x * m[:, t:t + 1], w_ref[t])
    if res:
        y += r_ref[...]
    y_ref[...] = y
    _accum_stats(so_ref, m, y)


def _stage(x, m, w, b, norm=None, res=None):
    n, fin = x.shape
    fout = w.shape[2]
    row = lambda i: (i, 0)
    full3 = pl.BlockSpec(None, lambda i: (0, 0, 0))
    full2 = pl.BlockSpec(None, lambda i: (0, 0))
    in_specs = [pl.BlockSpec((BN, fin), row), pl.BlockSpec((BN, T), row),
                full3, full2]
    args = [x, m, w, b]
    if norm is not None:
        st, g, be = norm
        in_specs += [full3, full2, full2]
        args += [st, g, be]
    if res is not None:
        in_specs.append(pl.BlockSpec((BN, fout), row))
        args.append(res)
    return pl.pallas_call(
        functools.partial(_stage_body, norm is not None, res is not None),
        grid=(NB,),
        in_specs=in_specs,
        out_specs=[pl.BlockSpec((BN, fout), row),
                   pl.BlockSpec((3, T, fout), lambda i: (0, 0, 0))],
        out_shape=[jax.ShapeDtypeStruct((n, fout), jnp.float32),
                   jax.ShapeDtypeStruct((3, T, fout), jnp.float32)],
    )(*args)


def _rgcn_dense_body(h_ref, m_ref, st_ref, g_ref, be_ref, relw_ref, rootw_ref,
                     tab_ref, root_ref):
    m = m_ref[...]
    sc, sh = _norm_coefs(st_ref[...], g_ref[...], be_ref[...])
    y = _leaky(h_ref[...] * _dot(m, sc) + _dot(m, sh))
    for r in range(R):
        tab_ref[r] = _dot(y, relw_ref[r])
    root_ref[...] = _dot(y, rootw_ref[...])


def _rgcn_dense(h, m, st, g, be, relw, rootw):
    row = lambda i: (i, 0)
    full3 = pl.BlockSpec(None, lambda i: (0, 0, 0))
    full2 = pl.BlockSpec(None, lambda i: (0, 0))
    return pl.pallas_call(
        _rgcn_dense_body,
        grid=(NB,),
        in_specs=[pl.BlockSpec((BN, C), row), pl.BlockSpec((BN, T), row),
                  full3, full2, full2, full3, full2],
        out_specs=[pl.BlockSpec((R, BN, C), lambda i: (0, i, 0)),
                   pl.BlockSpec((BN, C), row)],
        out_shape=[jax.ShapeDtypeStruct((R, N, C), jnp.float32),
                   jax.ShapeDtypeStruct((N, C), jnp.float32)],
    )(h, m, st, g, be, relw, rootw)


def _combine_body(h_ref, p_ref, root_ref, b_ref, m_ref, y_ref, so_ref):
    y = h_ref[...] + p_ref[0] + p_ref[1] + root_ref[...] + b_ref[...]
    y_ref[...] = y
    _accum_stats(so_ref, m_ref[...], y)


def _combine(h, parts, root, bias, m):
    row = lambda i: (i, 0)
    return pl.pallas_call(
        _combine_body,
        grid=(NB,),
        in_specs=[pl.BlockSpec((BN, C), row),
                  pl.BlockSpec((2, BN, C), lambda i: (0, i, 0)),
                  pl.BlockSpec((BN, C), row),
                  pl.BlockSpec(None, lambda i: (0, 0)),
                  pl.BlockSpec((BN, T), row)],
        out_specs=[pl.BlockSpec((BN, C), row),
                   pl.BlockSpec((3, T, C), lambda i: (0, 0, 0))],
        out_shape=[jax.ShapeDtypeStruct((N, C), jnp.float32),
                   jax.ShapeDtypeStruct((3, T, C), jnp.float32)],
    )(h, parts, root, bias, m)


def _prep_body(et_ref, src_ref, nt_ref, idx_ref, m_ref):
    idx_ref[...] = et_ref[...] * N + src_ref[...]
    tt = jax.lax.broadcasted_iota(jnp.int32, (N, T), 1)
    m_ref[...] = (nt_ref[...] == tt).astype(jnp.float32)


def _prep(et2, src2, nt):
    return pl.pallas_call(
        _prep_body,
        out_shape=[jax.ShapeDtypeStruct((E_PAD // CHUNK, CHUNK), jnp.int32),
                   jax.ShapeDtypeStruct((N, T), jnp.float32)],
    )(et2, src2, nt)


def _sc_rgcn_body(tab_ref, idx_ref, dst_ref, zero_ref, out_ref,
                  idx_v, dst_v, rows_v, acc, sem):
    c = lax.axis_index("c")
    s = lax.axis_index("s")
    w = s * 2 + c

    pltpu.sync_copy(zero_ref.at[pl.ds(s * RPS, RPS)], acc.at[pl.ds(s * RPS, RPS)])
    pltpu.sync_copy(idx_ref.at[pl.ds(w * CPW, CPW)], idx_v)
    pltpu.sync_copy(dst_ref.at[pl.ds(w * CPW, CPW)], dst_v)
    plsc.subcore_barrier()

    def body(j, carry):
        pltpu.async_copy(tab_ref.at[idx_v.at[j]], rows_v, sem).wait()
        pltpu.sync_copy(rows_v, acc.at[dst_v.at[j]], add=True)
        return carry

    lax.fori_loop(0, CPW, body, 0, unroll=False)
    plsc.subcore_barrier()
    pltpu.sync_copy(acc.at[pl.ds(s * RPS, RPS)],
                    out_ref.at[c].at[pl.ds(s * RPS, RPS)])


@functools.partial(
    pl.kernel,
    out_type=jax.ShapeDtypeStruct((2, N_PAD, C), jnp.float32),
    mesh=plsc.VectorSubcoreMesh(core_axis_name="c", subcore_axis_name="s"),
    scratch_types=[
        pltpu.VMEM((CPW, CHUNK), jnp.int32),
        pltpu.VMEM((CPW, CHUNK), jnp.int32),
        pltpu.VMEM((CHUNK, C), jnp.float32),
        pltpu.VMEM_SHARED((N_PAD, C), jnp.float32),
        pltpu.SemaphoreType.DMA,
    ],
)
def _sc_rgcn(tab_ref, idx_ref, dst_ref, zero_ref, out_ref,
             idx_v, dst_v, rows_v, acc, sem):
    _sc_rgcn_body(tab_ref, idx_ref, dst_ref, zero_ref, out_ref,
                  idx_v, dst_v, rows_v, acc, sem)


def kernel(x, params, edge_index, node_type, edge_type):
    p = params
    f32, i32 = jnp.float32, jnp.int32

    pad = E_PAD - E
    src2 = jnp.concatenate([edge_index[0], jnp.zeros((pad,), i32)]
                           ).reshape(E_PAD // CHUNK, CHUNK)
    dst2 = jnp.concatenate([edge_index[1], jnp.full((pad,), N, i32)]
                           ).reshape(E_PAD // CHUNK, CHUNK)
    et2 = jnp.concatenate([edge_type, jnp.zeros((pad,), i32)]
                          ).reshape(E_PAD // CHUNK, CHUNK)
    zeros = jnp.zeros((N_PAD, C), f32)

    idx2, m = _prep(et2, src2, node_type.reshape(N, 1))

    s0, st0 = _stage(x, m, p['ri_W0'], p['ri_b0'])
    h, st_h = _stage(s0, m, p['ri_W1'], p['ri_b1'],
                     norm=(st0, p['ri_g0'], p['ri_be0']))

    for blk in p['blocks']:
        tabs, root = _rgcn_dense(h, m, st_h, blk['cn_g'], blk['cn_b'],
                                 blk['rel_W'], blk['root_W'])
        parts = _sc_rgcn(tabs.reshape(R * N, C), idx2, dst2, zeros)
        h, st_h = _combine(h, parts, root, blk['conv_bias'].reshape(1, C), m)
        z1, st_z1 = _stage(h, m, blk['m_W0'], blk['m_b0'],
                           norm=(st_h, blk['mn_g'], blk['mn_b']))
        h, st_h = _stage(z1, m, blk['m_W1'], blk['m_b1'],
                         norm=(st_z1, blk['m_g0'], blk['m_be0']), res=h)

    u, st_u = _stage(h, m, p['ro_W0'], p['ro_b0'])
    out, _ = _stage(u, m, p['ro_W1'], p['ro_b1'],
                    norm=(st_u, p['ro_g0'], p['ro_be0']))
    return out


# speedup vs baseline: 4.6014x; 4.6014x over previous
import functools

import jax
import jax.numpy as jnp
from jax import lax
from jax.experimental import pallas as pl
from jax.experimental.pallas import tpu as pltpu
from jax.experimental.pallas import tpu_sc as plsc

N, E, D, T, R, C, H, OUT = 10000, 320000, 128, 3, 4, 32, 256, 32

NB = 10
BN = N // NB
NW = 32
CHUNK = 128
E_PAD = 327680
CPW = E_PAD // (NW * CHUNK)
N_PAD = 10112
RPS = N_PAD // 16

_P = jax.lax.Precision.HIGHEST


def _dot(a, b):
    return jnp.dot(a, b, precision=_P, preferred_element_type=jnp.float32)


def _leaky(v):
    return jnp.where(v >= 0, v, 0.01 * v)


def _norm_coefs(st, g, be):
    cnt = jnp.maximum(st[2], 1.0)
    mean = st[0] / cnt
    var = jnp.maximum(st[1] / cnt - mean * mean, 0.0)
    sc = jax.lax.rsqrt(var + 1e-5) * g
    return sc, be - mean * sc


def _accum_stats(so_ref, m, y):
    @pl.when(pl.program_id(0) == 0)
    def _():
        so_ref[...] = jnp.zeros_like(so_ref)

    mt = m.T
    so_ref[0] += _dot(mt, y)
    so_ref[1] += _dot(mt, y * y)
    so_ref[2] += _dot(mt, jnp.ones_like(y))


def _stage_body(norm, res, x_ref, m_ref, w_ref, b_ref, *rest):
    i = 0
    if norm:
        st_ref, g_ref, be_ref = rest[i:i + 3]
        i += 3
    if res:
        r_ref = rest[i]
        i += 1
    y_ref, so_ref = rest[i:i + 2]

    x = x_ref[...]
    m = m_ref[...]
    if norm:
        sc, sh = _norm_coefs(st_ref[...], g_ref[...], be_ref[...])
        x = _leaky(x * _dot(m, sc) + _dot(m, sh))
    y = _dot(m, b_ref[...])
    for t in range(T):
        y += _dot(x * m[:, t:t + 1], w_ref[t])
    if res:
        y += r_ref[...]
    y_ref[...] = y
    _accum_stats(so_ref, m, y)


def _stage(x, m, w, b, norm=None, res=None):
    n, fin = x.shape
    fout = w.shape[2]
    row = lambda i: (i, 0)
    full3 = pl.BlockSpec(None, lambda i: (0, 0, 0))
    full2 = pl.BlockSpec(None, lambda i: (0, 0))
    in_specs = [pl.BlockSpec((BN, fin), row), pl.BlockSpec((BN, T), row),
                full3, full2]
    args = [x, m, w, b]
    if norm is not None:
        st, g, be = norm
        in_specs += [full3, full2, full2]
        args += [st, g, be]
    if res is not None:
        in_specs.append(pl.BlockSpec((BN, fout), row))
        args.append(res)
    return pl.pallas_call(
        functools.partial(_stage_body, norm is not None, res is not None),
        grid=(NB,),
        in_specs=in_specs,
        out_specs=[pl.BlockSpec((BN, fout), row),
                   pl.BlockSpec((3, T, fout), lambda i: (0, 0, 0))],
        out_shape=[jax.ShapeDtypeStruct((n, fout), jnp.float32),
                   jax.ShapeDtypeStruct((3, T, fout), jnp.float32)],
    )(*args)


def _rgcn_dense_body(h_ref, m_ref, st_ref, g_ref, be_ref, relw_ref, rootw_ref,
                     tab_ref, root_ref):
    m = m_ref[...]
    sc, sh = _norm_coefs(st_ref[...], g_ref[...], be_ref[...])
    y = _leaky(h_ref[...] * _dot(m, sc) + _dot(m, sh))
    for r in range(R):
        tab_ref[r] = _dot(y, relw_ref[r])
    root_ref[...] = _dot(y, rootw_ref[...])


def _rgcn_dense(h, m, st, g, be, relw, rootw):
    row = lambda i: (i, 0)
    full3 = pl.BlockSpec(None, lambda i: (0, 0, 0))
    full2 = pl.BlockSpec(None, lambda i: (0, 0))
    return pl.pallas_call(
        _rgcn_dense_body,
        grid=(NB,),
        in_specs=[pl.BlockSpec((BN, C), row), pl.BlockSpec((BN, T), row),
                  full3, full2, full2, full3, full2],
        out_specs=[pl.BlockSpec((R, BN, C), lambda i: (0, i, 0)),
                   pl.BlockSpec((BN, C), row)],
        out_shape=[jax.ShapeDtypeStruct((R, N, C), jnp.float32),
                   jax.ShapeDtypeStruct((N, C), jnp.float32)],
    )(h, m, st, g, be, relw, rootw)


def _combine_body(h_ref, p_ref, root_ref, b_ref, m_ref, y_ref, so_ref):
    y = h_ref[...] + p_ref[0] + p_ref[1] + root_ref[...] + b_ref[...]
    y_ref[...] = y
    _accum_stats(so_ref, m_ref[...], y)


def _combine(h, parts, root, bias, m):
    row = lambda i: (i, 0)
    return pl.pallas_call(
        _combine_body,
        grid=(NB,),
        in_specs=[pl.BlockSpec((BN, C), row),
                  pl.BlockSpec((2, BN, C), lambda i: (0, i, 0)),
                  pl.BlockSpec((BN, C), row),
                  pl.BlockSpec(None, lambda i: (0, 0)),
                  pl.BlockSpec((BN, T), row)],
        out_specs=[pl.BlockSpec((BN, C), row),
                   pl.BlockSpec((3, T, C), lambda i: (0, 0, 0))],
        out_shape=[jax.ShapeDtypeStruct((N, C), jnp.float32),
                   jax.ShapeDtypeStruct((3, T, C), jnp.float32)],
    )(h, parts, root, bias, m)


def _prep_body(et_ref, src_ref, nt_ref, idx_ref, m_ref):
    idx_ref[...] = et_ref[...] * N + src_ref[...]
    tt = jax.lax.broadcasted_iota(jnp.int32, (N, T), 1)
    m_ref[...] = (nt_ref[...] == tt).astype(jnp.float32)


def _prep(et2, src2, nt):
    return pl.pallas_call(
        _prep_body,
        out_shape=[jax.ShapeDtypeStruct((E_PAD // CHUNK, CHUNK), jnp.int32),
                   jax.ShapeDtypeStruct((N, T), jnp.float32)],
    )(et2, src2, nt)


def _sc_rgcn_body(tab_ref, idx_ref, dst_ref, zero_ref, out_ref,
                  idx_v, dst_v, rows_v, acc, sem):
    c = lax.axis_index("c")
    s = lax.axis_index("s")
    w = s * 2 + c

    pltpu.sync_copy(zero_ref.at[pl.ds(s * RPS, RPS)], acc.at[pl.ds(s * RPS, RPS)])
    pltpu.sync_copy(idx_ref.at[pl.ds(w * CPW, CPW)], idx_v)
    pltpu.sync_copy(dst_ref.at[pl.ds(w * CPW, CPW)], dst_v)
    plsc.subcore_barrier()

    def body(j, carry):
        pltpu.async_copy(tab_ref.at[idx_v.at[j]], rows_v, sem).wait()
        pltpu.sync_copy(rows_v, acc.at[dst_v.at[j]], add=True)
        return carry

    lax.fori_loop(0, CPW, body, 0, unroll=False)
    plsc.subcore_barrier()
    pltpu.sync_copy(acc.at[pl.ds(s * RPS, RPS)],
                    out_ref.at[c].at[pl.ds(s * RPS, RPS)])


@functools.cache
def _sc_rgcn_kernel():
    return pl.kernel(
        _sc_rgcn_body,
        out_type=jax.ShapeDtypeStruct((2, N_PAD, C), jnp.float32),
        mesh=plsc.VectorSubcoreMesh(core_axis_name="c", subcore_axis_name="s"),
        scratch_types=[
            pltpu.VMEM((CPW, CHUNK), jnp.int32),
            pltpu.VMEM((CPW, CHUNK), jnp.int32),
            pltpu.VMEM((CHUNK, C), jnp.float32),
            pltpu.VMEM_SHARED((N_PAD, C), jnp.float32),
            pltpu.SemaphoreType.DMA,
        ],
        compiler_params=pltpu.CompilerParams(use_tc_tiling_on_sc=False),
    )


def _sc_rgcn(tabs_flat, idx2, dst2, zeros):
    return _sc_rgcn_kernel()(tabs_flat, idx2, dst2, zeros)


def kernel(x, params, edge_index, node_type, edge_type):
    p = params
    f32, i32 = jnp.float32, jnp.int32

    pad = E_PAD - E
    src2 = jnp.concatenate([edge_index[0], jnp.zeros((pad,), i32)]
                           ).reshape(E_PAD // CHUNK, CHUNK)
    dst2 = jnp.concatenate([edge_index[1], jnp.full((pad,), N, i32)]
                           ).reshape(E_PAD // CHUNK, CHUNK)
    et2 = jnp.concatenate([edge_type, jnp.zeros((pad,), i32)]
                          ).reshape(E_PAD // CHUNK, CHUNK)
    zeros = jnp.zeros((N_PAD, C), f32)

    idx2, m = _prep(et2, src2, node_type.reshape(N, 1))

    s0, st0 = _stage(x, m, p['ri_W0'], p['ri_b0'])
    h, st_h = _stage(s0, m, p['ri_W1'], p['ri_b1'],
                     norm=(st0, p['ri_g0'], p['ri_be0']))

    for blk in p['blocks']:
        tabs, root = _rgcn_dense(h, m, st_h, blk['cn_g'], blk['cn_b'],
                                 blk['rel_W'], blk['root_W'])
        parts = _sc_rgcn(tabs.reshape(R * N, C), idx2, dst2, zeros)
        h, st_h = _combine(h, parts, root, blk['conv_bias'].reshape(1, C), m)
        z1, st_z1 = _stage(h, m, blk['m_W0'], blk['m_b0'],
                           norm=(st_h, blk['mn_g'], blk['mn_b']))
        h, st_h = _stage(z1, m, blk['m_W1'], blk['m_b1'],
                         norm=(st_z1, blk['m_g0'], blk['m_be0']), res=h)

    u, st_u = _stage(h, m, p['ro_W0'], p['ro_b0'])
    out, _ = _stage(u, m, p['ro_W1'], p['ro_b1'],
                    norm=(st_u, p['ro_g0'], p['ro_be0']))
    return out


# speedup vs baseline: 6.7198x; 1.4604x over previous
import functools

import jax
import jax.numpy as jnp
from jax import lax
from jax.experimental import pallas as pl
from jax.experimental.pallas import tpu as pltpu
from jax.experimental.pallas import tpu_sc as plsc

N, E, D, T, R, C, H, OUT = 10000, 320000, 128, 3, 4, 32, 256, 32

NB = 5
BN = N // NB
NW = 32
CHUNK = 128
E_PAD = 327680
CPW = E_PAD // (NW * CHUNK)
N_PAD = 10112
RPS = N_PAD // 16

def _dot(a, b):
    f32, bf16 = jnp.float32, jnp.bfloat16
    ah = a.astype(bf16)
    al = (a - ah.astype(f32)).astype(bf16)
    bh = b.astype(bf16)
    bl = (b - bh.astype(f32)).astype(bf16)

    def d(u, v):
        return jax.lax.dot_general(u, v, (((u.ndim - 1,), (0,)), ((), ())),
                                   preferred_element_type=f32)

    return d(ah, bh) + (d(al, bh) + d(ah, bl))


def _leaky(v):
    return jnp.where(v >= 0, v, 0.01 * v)


def _norm_coefs(st, g, be):
    cnt = jnp.maximum(st[2], 1.0)
    mean = st[0] / cnt
    var = jnp.maximum(st[1] / cnt - mean * mean, 0.0)
    sc = jax.lax.rsqrt(var + 1e-5) * g
    return sc, be - mean * sc


def _accum_stats(so_ref, m, y):
    @pl.when(pl.program_id(0) == 0)
    def _():
        so_ref[...] = jnp.zeros_like(so_ref)

    mt = m.T
    so_ref[0] += _dot(mt, y)
    so_ref[1] += _dot(mt, y * y)
    so_ref[2] += _dot(mt, jnp.ones_like(y))


def _stage_body(norm, res, x_ref, m_ref, w_ref, b_ref, *rest):
    i = 0
    if norm:
        st_ref, g_ref, be_ref = rest[i:i + 3]
        i += 3
    if res:
        r_ref = rest[i]
        i += 1
    y_ref, so_ref = rest[i:i + 2]

    x = x_ref[...]
    m = m_ref[...]
    if norm:
        sc, sh = _norm_coefs(st_ref[...], g_ref[...], be_ref[...])
        x = _leaky(x * _dot(m, sc) + _dot(m, sh))
    y = _dot(m, b_ref[...])
    for t in range(T):
        y += _dot(x * m[:, t:t + 1], w_ref[t])
    if res:
        y += r_ref[...]
    y_ref[...] = y
    _accum_stats(so_ref, m, y)


def _stage(x, m, w, b, norm=None, res=None):
    n, fin = x.shape
    fout = w.shape[2]
    row = lambda i: (i, 0)
    full3 = pl.BlockSpec(None, lambda i: (0, 0, 0))
    full2 = pl.BlockSpec(None, lambda i: (0, 0))
    in_specs = [pl.BlockSpec((BN, fin), row), pl.BlockSpec((BN, T), row),
                full3, full2]
    args = [x, m, w, b]
    if norm is not None:
        st, g, be = norm
        in_specs += [full3, full2, full2]
        args += [st, g, be]
    if res is not None:
        in_specs.append(pl.BlockSpec((BN, fout), row))
        args.append(res)
    return pl.pallas_call(
        functools.partial(_stage_body, norm is not None, res is not None),
        grid=(NB,),
        in_specs=in_specs,
        out_specs=[pl.BlockSpec((BN, fout), row),
                   pl.BlockSpec((3, T, fout), lambda i: (0, 0, 0))],
        out_shape=[jax.ShapeDtypeStruct((n, fout), jnp.float32),
                   jax.ShapeDtypeStruct((3, T, fout), jnp.float32)],
    )(*args)


def _rgcn_dense_body(h_ref, m_ref, st_ref, g_ref, be_ref, relw_ref, rootw_ref,
                     tab_ref, root_ref):
    m = m_ref[...]
    sc, sh = _norm_coefs(st_ref[...], g_ref[...], be_ref[...])
    y = _leaky(h_ref[...] * _dot(m, sc) + _dot(m, sh))
    for r in range(R):
        tab_ref[r] = _dot(y, relw_ref[r])
    root_ref[...] = _dot(y, rootw_ref[...])


def _rgcn_dense(h, m, st, g, be, relw, rootw):
    row = lambda i: (i, 0)
    full3 = pl.BlockSpec(None, lambda i: (0, 0, 0))
    full2 = pl.BlockSpec(None, lambda i: (0, 0))
    return pl.pallas_call(
        _rgcn_dense_body,
        grid=(NB,),
        in_specs=[pl.BlockSpec((BN, C), row), pl.BlockSpec((BN, T), row),
                  full3, full2, full2, full3, full2],
        out_specs=[pl.BlockSpec((R, BN, C), lambda i: (0, i, 0)),
                   pl.BlockSpec((BN, C), row)],
        out_shape=[jax.ShapeDtypeStruct((R, N, C), jnp.float32),
                   jax.ShapeDtypeStruct((N, C), jnp.float32)],
    )(h, m, st, g, be, relw, rootw)


def _combine_body(h_ref, p_ref, root_ref, b_ref, m_ref, y_ref, so_ref):
    y = h_ref[...] + p_ref[0] + p_ref[1] + root_ref[...] + b_ref[...]
    y_ref[...] = y
    _accum_stats(so_ref, m_ref[...], y)


def _combine(h, parts, root, bias, m):
    row = lambda i: (i, 0)
    return pl.pallas_call(
        _combine_body,
        grid=(NB,),
        in_specs=[pl.BlockSpec((BN, C), row),
                  pl.BlockSpec((2, BN, C), lambda i: (0, i, 0)),
                  pl.BlockSpec((BN, C), row),
                  pl.BlockSpec(None, lambda i: (0, 0)),
                  pl.BlockSpec((BN, T), row)],
        out_specs=[pl.BlockSpec((BN, C), row),
                   pl.BlockSpec((3, T, C), lambda i: (0, 0, 0))],
        out_shape=[jax.ShapeDtypeStruct((N, C), jnp.float32),
                   jax.ShapeDtypeStruct((3, T, C), jnp.float32)],
    )(h, parts, root, bias, m)


def _prep_body(et_ref, src_ref, nt_ref, idx_ref, m_ref):
    idx_ref[...] = et_ref[...] * N + src_ref[...]
    tt = jax.lax.broadcasted_iota(jnp.int32, (N, T), 1)
    m_ref[...] = (nt_ref[...] == tt).astype(jnp.float32)


def _prep(et2, src2, nt):
    return pl.pallas_call(
        _prep_body,
        out_shape=[jax.ShapeDtypeStruct((E_PAD // CHUNK, CHUNK), jnp.int32),
                   jax.ShapeDtypeStruct((N, T), jnp.float32)],
    )(et2, src2, nt)


def _sc_rgcn_body(tab_ref, idx_ref, dst_ref, zero_ref, out_ref,
                  idx_v, dst_v, rows_v, acc, sem):
    c = lax.axis_index("c")
    s = lax.axis_index("s")
    w = s * 2 + c

    pltpu.sync_copy(zero_ref.at[pl.ds(s * RPS, RPS)], acc.at[pl.ds(s * RPS, RPS)])
    pltpu.sync_copy(idx_ref.at[pl.ds(w * CPW, CPW)], idx_v)
    pltpu.sync_copy(dst_ref.at[pl.ds(w * CPW, CPW)], dst_v)
    plsc.subcore_barrier()

    def body(j, carry):
        pltpu.async_copy(tab_ref.at[idx_v.at[j]], rows_v, sem).wait()
        pltpu.sync_copy(rows_v, acc.at[dst_v.at[j]], add=True)
        return carry

    lax.fori_loop(0, CPW, body, 0, unroll=False)
    plsc.subcore_barrier()
    pltpu.sync_copy(acc.at[pl.ds(s * RPS, RPS)],
                    out_ref.at[c].at[pl.ds(s * RPS, RPS)])


@functools.cache
def _sc_rgcn_kernel():
    return pl.kernel(
        _sc_rgcn_body,
        out_type=jax.ShapeDtypeStruct((2, N_PAD, C), jnp.float32),
        mesh=plsc.VectorSubcoreMesh(core_axis_name="c", subcore_axis_name="s"),
        scratch_types=[
            pltpu.VMEM((CPW, CHUNK), jnp.int32),
            pltpu.VMEM((CPW, CHUNK), jnp.int32),
            pltpu.VMEM((CHUNK, C), jnp.float32),
            pltpu.VMEM_SHARED((N_PAD, C), jnp.float32),
            pltpu.SemaphoreType.DMA,
        ],
        compiler_params=pltpu.CompilerParams(use_tc_tiling_on_sc=False),
    )


def _sc_rgcn(tabs_flat, idx2, dst2, zeros):
    return _sc_rgcn_kernel()(tabs_flat, idx2, dst2, zeros)


def kernel(x, params, edge_index, node_type, edge_type):
    p = params
    f32, i32 = jnp.float32, jnp.int32

    pad = E_PAD - E
    src2 = jnp.concatenate([edge_index[0], jnp.zeros((pad,), i32)]
                           ).reshape(E_PAD // CHUNK, CHUNK)
    dst2 = jnp.concatenate([edge_index[1], jnp.full((pad,), N, i32)]
                           ).reshape(E_PAD // CHUNK, CHUNK)
    et2 = jnp.concatenate([edge_type, jnp.zeros((pad,), i32)]
                          ).reshape(E_PAD // CHUNK, CHUNK)
    zeros = jnp.zeros((N_PAD, C), f32)

    idx2, m = _prep(et2, src2, node_type.reshape(N, 1))

    s0, st0 = _stage(x, m, p['ri_W0'], p['ri_b0'])
    h, st_h = _stage(s0, m, p['ri_W1'], p['ri_b1'],
                     norm=(st0, p['ri_g0'], p['ri_be0']))

    for blk in p['blocks']:
        tabs, root = _rgcn_dense(h, m, st_h, blk['cn_g'], blk['cn_b'],
                                 blk['rel_W'], blk['root_W'])
        parts = _sc_rgcn(tabs.reshape(R * N, C), idx2, dst2, zeros)
        h, st_h = _combine(h, parts, root, blk['conv_bias'].reshape(1, C), m)
        z1, st_z1 = _stage(h, m, blk['m_W0'], blk['m_b0'],
                           norm=(st_h, blk['mn_g'], blk['mn_b']))
        h, st_h = _stage(z1, m, blk['m_W1'], blk['m_b1'],
                         norm=(st_z1, blk['m_g0'], blk['m_be0']), res=h)

    u, st_u = _stage(h, m, p['ro_W0'], p['ro_b0'])
    out, _ = _stage(u, m, p['ro_W1'], p['ro_b1'],
                    norm=(st_u, p['ro_g0'], p['ro_be0']))
    return out


# speedup vs baseline: 6.7420x; 1.0033x over previous
import functools

import jax
import jax.numpy as jnp
from jax import lax
from jax.experimental import pallas as pl
from jax.experimental.pallas import tpu as pltpu
from jax.experimental.pallas import tpu_sc as plsc

N, E, D, T, R, C, H, OUT = 10000, 320000, 128, 3, 4, 32, 256, 32

NB = 5
BN = N // NB
NW = 32
CHUNK = 128
E_PAD = 327680
CPW0 = 56
CPW1 = 104
N_PAD = 10112
RPS = N_PAD // 16

def _dot(a, b):
    f32, bf16 = jnp.float32, jnp.bfloat16
    ah = a.astype(bf16)
    al = (a - ah.astype(f32)).astype(bf16)
    bh = b.astype(bf16)
    bl = (b - bh.astype(f32)).astype(bf16)

    def d(u, v):
        return jax.lax.dot_general(u, v, (((u.ndim - 1,), (0,)), ((), ())),
                                   preferred_element_type=f32)

    return d(ah, bh) + (d(al, bh) + d(ah, bl))


def _leaky(v):
    return jnp.where(v >= 0, v, 0.01 * v)


def _norm_coefs(st, g, be):
    cnt = jnp.maximum(st[2], 1.0)
    mean = st[0] / cnt
    var = jnp.maximum(st[1] / cnt - mean * mean, 0.0)
    sc = jax.lax.rsqrt(var + 1e-5) * g
    return sc, be - mean * sc


def _accum_stats(so_ref, m, y):
    @pl.when(pl.program_id(0) == 0)
    def _():
        so_ref[...] = jnp.zeros_like(so_ref)

    mt = m.T
    so_ref[0] += _dot(mt, y)
    so_ref[1] += _dot(mt, y * y)
    so_ref[2] += _dot(mt, jnp.ones_like(y))


def _stage_body(norm, res, x_ref, m_ref, w_ref, b_ref, *rest):
    i = 0
    if norm:
        st_ref, g_ref, be_ref = rest[i:i + 3]
        i += 3
    if res:
        r_ref = rest[i]
        i += 1
    y_ref, so_ref = rest[i:i + 2]

    x = x_ref[...]
    m = m_ref[...]
    if norm:
        sc, sh = _norm_coefs(st_ref[...], g_ref[...], be_ref[...])
        x = _leaky(x * _dot(m, sc) + _dot(m, sh))
    y = _dot(m, b_ref[...])
    for t in range(T):
        y += _dot(x * m[:, t:t + 1], w_ref[t])
    if res:
        y += r_ref[...]
    y_ref[...] = y
    _accum_stats(so_ref, m, y)


def _stage(x, m, w, b, norm=None, res=None):
    n, fin = x.shape
    fout = w.shape[2]
    row = lambda i: (i, 0)
    full3 = pl.BlockSpec(None, lambda i: (0, 0, 0))
    full2 = pl.BlockSpec(None, lambda i: (0, 0))
    in_specs = [pl.BlockSpec((BN, fin), row), pl.BlockSpec((BN, T), row),
                full3, full2]
    args = [x, m, w, b]
    if norm is not None:
        st, g, be = norm
        in_specs += [full3, full2, full2]
        args += [st, g, be]
    if res is not None:
        in_specs.append(pl.BlockSpec((BN, fout), row))
        args.append(res)
    return pl.pallas_call(
        functools.partial(_stage_body, norm is not None, res is not None),
        grid=(NB,),
        in_specs=in_specs,
        out_specs=[pl.BlockSpec((BN, fout), row),
                   pl.BlockSpec((3, T, fout), lambda i: (0, 0, 0))],
        out_shape=[jax.ShapeDtypeStruct((n, fout), jnp.float32),
                   jax.ShapeDtypeStruct((3, T, fout), jnp.float32)],
    )(*args)


def _rgcn_dense_body(h_ref, m_ref, st_ref, g_ref, be_ref, relwc_ref, rootw_ref,
                     tab_ref, root_ref):
    m = m_ref[...]
    sc, sh = _norm_coefs(st_ref[...], g_ref[...], be_ref[...])
    y = _leaky(h_ref[...] * _dot(m, sc) + _dot(m, sh))
    tab_ref[...] = _dot(y, relwc_ref[...])
    root_ref[...] = _dot(y, rootw_ref[...])


def _rgcn_dense(h, m, st, g, be, relwc, rootw):
    row = lambda i: (i, 0)
    full3 = pl.BlockSpec(None, lambda i: (0, 0, 0))
    full2 = pl.BlockSpec(None, lambda i: (0, 0))
    return pl.pallas_call(
        _rgcn_dense_body,
        grid=(NB,),
        in_specs=[pl.BlockSpec((BN, C), row), pl.BlockSpec((BN, T), row),
                  full3, full2, full2, full2, full2],
        out_specs=[pl.BlockSpec((BN, R * C), row),
                   pl.BlockSpec((BN, C), row)],
        out_shape=[jax.ShapeDtypeStruct((N, R * C), jnp.float32),
                   jax.ShapeDtypeStruct((N, C), jnp.float32)],
    )(h, m, st, g, be, relwc, rootw)


def _combine_body(h_ref, p_ref, root_ref, b_ref, m_ref, y_ref, so_ref):
    y = h_ref[...] + p_ref[0] + p_ref[1] + root_ref[...] + b_ref[...]
    y_ref[...] = y
    _accum_stats(so_ref, m_ref[...], y)


def _combine(h, parts, root, bias, m):
    row = lambda i: (i, 0)
    return pl.pallas_call(
        _combine_body,
        grid=(NB,),
        in_specs=[pl.BlockSpec((BN, C), row),
                  pl.BlockSpec((2, BN, C), lambda i: (0, i, 0)),
                  pl.BlockSpec((BN, C), row),
                  pl.BlockSpec(None, lambda i: (0, 0)),
                  pl.BlockSpec((BN, T), row)],
        out_specs=[pl.BlockSpec((BN, C), row),
                   pl.BlockSpec((3, T, C), lambda i: (0, 0, 0))],
        out_shape=[jax.ShapeDtypeStruct((N, C), jnp.float32),
                   jax.ShapeDtypeStruct((3, T, C), jnp.float32)],
    )(h, parts, root, bias, m)


def _prep_body(et_ref, src_ref, nt_ref, idx_ref, m_ref):
    idx_ref[...] = src_ref[...] * R + et_ref[...]
    tt = jax.lax.broadcasted_iota(jnp.int32, (N, T), 1)
    m_ref[...] = (nt_ref[...] == tt).astype(jnp.float32)


def _prep(et2, src2, nt):
    return pl.pallas_call(
        _prep_body,
        out_shape=[jax.ShapeDtypeStruct((E_PAD // CHUNK, CHUNK), jnp.int32),
                   jax.ShapeDtypeStruct((N, T), jnp.float32)],
    )(et2, src2, nt)


def _sc_rgcn_body(tab_ref, idx_ref, dst_ref, zero_ref, out_ref,
                  idx_v, dst_v, rows_v, acc, sem):
    c = lax.axis_index("c")
    s = lax.axis_index("s")

    pltpu.sync_copy(zero_ref.at[pl.ds(s * RPS, RPS)], acc.at[pl.ds(s * RPS, RPS)])

    @pl.when(c == 0)
    def _():
        pltpu.sync_copy(idx_ref.at[pl.ds(s * CPW0, CPW0)],
                        idx_v.at[pl.ds(0, CPW0)])
        pltpu.sync_copy(dst_ref.at[pl.ds(s * CPW0, CPW0)],
                        dst_v.at[pl.ds(0, CPW0)])

    @pl.when(c == 1)
    def _():
        pltpu.sync_copy(idx_ref.at[pl.ds(16 * CPW0 + s * CPW1, CPW1)], idx_v)
        pltpu.sync_copy(dst_ref.at[pl.ds(16 * CPW0 + s * CPW1, CPW1)], dst_v)

    plsc.subcore_barrier()
    n = jnp.where(c == 0, CPW0, CPW1)

    def body(j, carry):
        pltpu.async_copy(tab_ref.at[idx_v.at[j]], rows_v, sem).wait()
        pltpu.sync_copy(rows_v, acc.at[dst_v.at[j]], add=True)
        return carry

    lax.fori_loop(0, n, body, 0, unroll=False)
    plsc.subcore_barrier()
    pltpu.sync_copy(acc.at[pl.ds(s * RPS, RPS)],
                    out_ref.at[c].at[pl.ds(s * RPS, RPS)])


@functools.cache
def _sc_rgcn_kernel():
    return pl.kernel(
        _sc_rgcn_body,
        out_type=jax.ShapeDtypeStruct((2, N_PAD, C), jnp.float32),
        mesh=plsc.VectorSubcoreMesh(core_axis_name="c", subcore_axis_name="s"),
        scratch_types=[
            pltpu.VMEM((CPW1, CHUNK), jnp.int32),
            pltpu.VMEM((CPW1, CHUNK), jnp.int32),
            pltpu.VMEM((CHUNK, C), jnp.float32),
            pltpu.VMEM_SHARED((N_PAD, C), jnp.float32),
            pltpu.SemaphoreType.DMA,
        ],
        compiler_params=pltpu.CompilerParams(use_tc_tiling_on_sc=False),
    )


def _sc_rgcn(tabs_flat, idx2, dst2, zeros):
    return _sc_rgcn_kernel()(tabs_flat, idx2, dst2, zeros)


def kernel(x, params, edge_index, node_type, edge_type):
    p = params
    f32, i32 = jnp.float32, jnp.int32

    pad = E_PAD - E
    src2 = jnp.concatenate([edge_index[0], jnp.zeros((pad,), i32)]
                           ).reshape(E_PAD // CHUNK, CHUNK)
    dst2 = jnp.concatenate([edge_index[1], jnp.full((pad,), N, i32)]
                           ).reshape(E_PAD // CHUNK, CHUNK)
    et2 = jnp.concatenate([edge_type, jnp.zeros((pad,), i32)]
                          ).reshape(E_PAD // CHUNK, CHUNK)
    zeros = jnp.zeros((N_PAD, C), f32)

    idx2, m = _prep(et2, src2, node_type.reshape(N, 1))

    s0, st0 = _stage(x, m, p['ri_W0'], p['ri_b0'])
    h, st_h = _stage(s0, m, p['ri_W1'], p['ri_b1'],
                     norm=(st0, p['ri_g0'], p['ri_be0']))

    for blk in p['blocks']:
        relwc = blk['rel_W'].transpose(1, 0, 2).reshape(C, R * C)
        tabs, root = _rgcn_dense(h, m, st_h, blk['cn_g'], blk['cn_b'],
                                 relwc, blk['root_W'])
        parts = _sc_rgcn(tabs.reshape(R * N, C), idx2, dst2, zeros)
        h, st_h = _combine(h, parts, root, blk['conv_bias'].reshape(1, C), m)
        z1, st_z1 = _stage(h, m, blk['m_W0'], blk['m_b0'],
                           norm=(st_h, blk['mn_g'], blk['mn_b']))
        h, st_h = _stage(z1, m, blk['m_W1'], blk['m_b1'],
                         norm=(st_z1, blk['m_g0'], blk['m_be0']), res=h)

    u, st_u = _stage(h, m, p['ro_W0'], p['ro_b0'])
    out, _ = _stage(u, m, p['ro_W1'], p['ro_b1'],
                    norm=(st_u, p['ro_g0'], p['ro_be0']))
    return out


# speedup vs baseline: 7.5538x; 1.1204x over previous
import functools

import jax
import jax.numpy as jnp
from jax import lax
from jax.experimental import pallas as pl
from jax.experimental.pallas import tpu as pltpu
from jax.experimental.pallas import tpu_sc as plsc

N, E, D, T, R, C, H, OUT = 10000, 320000, 128, 3, 4, 32, 256, 32

NB = 5
BN = N // NB
NW = 32
CHUNK = 128
E_PAD = 327680
CPW0 = 104
CPW1 = 56
N_PAD = 10112
RPS = N_PAD // 16

def _dot(a, b):
    f32, bf16 = jnp.float32, jnp.bfloat16
    ah = a.astype(bf16)
    al = (a - ah.astype(f32)).astype(bf16)
    bh = b.astype(bf16)
    bl = (b - bh.astype(f32)).astype(bf16)

    def d(u, v):
        return jax.lax.dot_general(u, v, (((u.ndim - 1,), (0,)), ((), ())),
                                   preferred_element_type=f32)

    return d(ah, bh) + (d(al, bh) + d(ah, bl))


def _leaky(v):
    return jnp.where(v >= 0, v, 0.01 * v)


def _norm_coefs(st, g, be):
    cnt = jnp.maximum(st[2], 1.0)
    mean = st[0] / cnt
    var = jnp.maximum(st[1] / cnt - mean * mean, 0.0)
    sc = jax.lax.rsqrt(var + 1e-5) * g
    return sc, be - mean * sc


def _accum_stats(so_ref, m, y):
    @pl.when(pl.program_id(0) == 0)
    def _():
        so_ref[...] = jnp.zeros_like(so_ref)

    mt = m.T
    so_ref[0] += _dot(mt, y)
    so_ref[1] += _dot(mt, y * y)
    so_ref[2] += _dot(mt, jnp.ones_like(y))


def _stage_body(norm, res, x_ref, m_ref, w_ref, b_ref, *rest):
    i = 0
    if norm:
        st_ref, g_ref, be_ref = rest[i:i + 3]
        i += 3
    if res:
        r_ref = rest[i]
        i += 1
    y_ref, so_ref = rest[i:i + 2]

    x = x_ref[...]
    m = m_ref[...]
    if norm:
        sc, sh = _norm_coefs(st_ref[...], g_ref[...], be_ref[...])
        x = _leaky(x * _dot(m, sc) + _dot(m, sh))
    y = _dot(m, b_ref[...])
    for t in range(T):
        y += _dot(x * m[:, t:t + 1], w_ref[t])
    if res:
        y += r_ref[...]
    y_ref[...] = y
    _accum_stats(so_ref, m, y)


def _stage(x, m, w, b, norm=None, res=None):
    n, fin = x.shape
    fout = w.shape[2]
    row = lambda i: (i, 0)
    full3 = pl.BlockSpec(None, lambda i: (0, 0, 0))
    full2 = pl.BlockSpec(None, lambda i: (0, 0))
    in_specs = [pl.BlockSpec((BN, fin), row), pl.BlockSpec((BN, T), row),
                full3, full2]
    args = [x, m, w, b]
    if norm is not None:
        st, g, be = norm
        in_specs += [full3, full2, full2]
        args += [st, g, be]
    if res is not None:
        in_specs.append(pl.BlockSpec((BN, fout), row))
        args.append(res)
    return pl.pallas_call(
        functools.partial(_stage_body, norm is not None, res is not None),
        grid=(NB,),
        in_specs=in_specs,
        out_specs=[pl.BlockSpec((BN, fout), row),
                   pl.BlockSpec((3, T, fout), lambda i: (0, 0, 0))],
        out_shape=[jax.ShapeDtypeStruct((n, fout), jnp.float32),
                   jax.ShapeDtypeStruct((3, T, fout), jnp.float32)],
    )(*args)


def _rgcn_dense_body(h_ref, m_ref, st_ref, g_ref, be_ref, relwc_ref, rootw_ref,
                     tab_ref, root_ref):
    m = m_ref[...]
    sc, sh = _norm_coefs(st_ref[...], g_ref[...], be_ref[...])
    y = _leaky(h_ref[...] * _dot(m, sc) + _dot(m, sh))
    tab_ref[...] = _dot(y, relwc_ref[...])
    root_ref[...] = _dot(y, rootw_ref[...])


def _rgcn_dense(h, m, st, g, be, relwc, rootw):
    row = lambda i: (i, 0)
    full3 = pl.BlockSpec(None, lambda i: (0, 0, 0))
    full2 = pl.BlockSpec(None, lambda i: (0, 0))
    return pl.pallas_call(
        _rgcn_dense_body,
        grid=(NB,),
        in_specs=[pl.BlockSpec((BN, C), row), pl.BlockSpec((BN, T), row),
                  full3, full2, full2, full2, full2],
        out_specs=[pl.BlockSpec((BN, R * C), row),
                   pl.BlockSpec((BN, C), row)],
        out_shape=[jax.ShapeDtypeStruct((N, R * C), jnp.float32),
                   jax.ShapeDtypeStruct((N, C), jnp.float32)],
    )(h, m, st, g, be, relwc, rootw)


def _combine_body(h_ref, p_ref, root_ref, b_ref, m_ref, y_ref, so_ref):
    y = h_ref[...] + p_ref[0] + p_ref[1] + root_ref[...] + b_ref[...]
    y_ref[...] = y
    _accum_stats(so_ref, m_ref[...], y)


def _combine(h, parts, root, bias, m):
    row = lambda i: (i, 0)
    return pl.pallas_call(
        _combine_body,
        grid=(NB,),
        in_specs=[pl.BlockSpec((BN, C), row),
                  pl.BlockSpec((2, BN, C), lambda i: (0, i, 0)),
                  pl.BlockSpec((BN, C), row),
                  pl.BlockSpec(None, lambda i: (0, 0)),
                  pl.BlockSpec((BN, T), row)],
        out_specs=[pl.BlockSpec((BN, C), row),
                   pl.BlockSpec((3, T, C), lambda i: (0, 0, 0))],
        out_shape=[jax.ShapeDtypeStruct((N, C), jnp.float32),
                   jax.ShapeDtypeStruct((3, T, C), jnp.float32)],
    )(h, parts, root, bias, m)


def _prep_body(et_ref, src_ref, nt_ref, idx_ref, m_ref):
    idx_ref[...] = src_ref[...] * R + et_ref[...]
    tt = jax.lax.broadcasted_iota(jnp.int32, (N, T), 1)
    m_ref[...] = (nt_ref[...] == tt).astype(jnp.float32)


def _prep(et2, src2, nt):
    return pl.pallas_call(
        _prep_body,
        out_shape=[jax.ShapeDtypeStruct((E_PAD // CHUNK, CHUNK), jnp.int32),
                   jax.ShapeDtypeStruct((N, T), jnp.float32)],
    )(et2, src2, nt)


def _sc_rgcn_body(tab_ref, idx_ref, dst_ref, zero_ref, out_ref,
                  idx_v, dst_v, rows_v, acc, sem):
    c = lax.axis_index("c")
    s = lax.axis_index("s")

    pltpu.sync_copy(zero_ref.at[pl.ds(s * RPS, RPS)], acc.at[pl.ds(s * RPS, RPS)])

    @pl.when(c == 0)
    def _():
        pltpu.sync_copy(idx_ref.at[pl.ds(s * CPW0, CPW0)],
                        idx_v.at[pl.ds(0, CPW0)])
        pltpu.sync_copy(dst_ref.at[pl.ds(s * CPW0, CPW0)],
                        dst_v.at[pl.ds(0, CPW0)])

    @pl.when(c == 1)
    def _():
        pltpu.sync_copy(idx_ref.at[pl.ds(16 * CPW0 + s * CPW1, CPW1)],
                        idx_v.at[pl.ds(0, CPW1)])
        pltpu.sync_copy(dst_ref.at[pl.ds(16 * CPW0 + s * CPW1, CPW1)],
                        dst_v.at[pl.ds(0, CPW1)])

    plsc.subcore_barrier()
    n = jnp.where(c == 0, CPW0, CPW1)

    def body(j, carry):
        pltpu.async_copy(tab_ref.at[idx_v.at[j]], rows_v, sem).wait()
        pltpu.sync_copy(rows_v, acc.at[dst_v.at[j]], add=True)
        return carry

    lax.fori_loop(0, n, body, 0, unroll=False)
    plsc.subcore_barrier()
    pltpu.sync_copy(acc.at[pl.ds(s * RPS, RPS)],
                    out_ref.at[c].at[pl.ds(s * RPS, RPS)])


@functools.cache
def _sc_rgcn_kernel():
    return pl.kernel(
        _sc_rgcn_body,
        out_type=jax.ShapeDtypeStruct((2, N_PAD, C), jnp.float32),
        mesh=plsc.VectorSubcoreMesh(core_axis_name="c", subcore_axis_name="s"),
        scratch_types=[
            pltpu.VMEM((max(CPW0, CPW1), CHUNK), jnp.int32),
            pltpu.VMEM((max(CPW0, CPW1), CHUNK), jnp.int32),
            pltpu.VMEM((CHUNK, C), jnp.float32),
            pltpu.VMEM_SHARED((N_PAD, C), jnp.float32),
            pltpu.SemaphoreType.DMA,
        ],
        compiler_params=pltpu.CompilerParams(use_tc_tiling_on_sc=False),
    )


def _sc_rgcn(tabs_flat, idx2, dst2, zeros):
    return _sc_rgcn_kernel()(tabs_flat, idx2, dst2, zeros)


def kernel(x, params, edge_index, node_type, edge_type):
    p = params
    f32, i32 = jnp.float32, jnp.int32

    pad = E_PAD - E
    src2 = jnp.concatenate([edge_index[0], jnp.zeros((pad,), i32)]
                           ).reshape(E_PAD // CHUNK, CHUNK)
    dst2 = jnp.concatenate([edge_index[1], jnp.full((pad,), N, i32)]
                           ).reshape(E_PAD // CHUNK, CHUNK)
    et2 = jnp.concatenate([edge_type, jnp.zeros((pad,), i32)]
                          ).reshape(E_PAD // CHUNK, CHUNK)
    zeros = jnp.zeros((N_PAD, C), f32)

    idx2, m = _prep(et2, src2, node_type.reshape(N, 1))

    s0, st0 = _stage(x, m, p['ri_W0'], p['ri_b0'])
    h, st_h = _stage(s0, m, p['ri_W1'], p['ri_b1'],
                     norm=(st0, p['ri_g0'], p['ri_be0']))

    for blk in p['blocks']:
        relwc = blk['rel_W'].transpose(1, 0, 2).reshape(C, R * C)
        tabs, root = _rgcn_dense(h, m, st_h, blk['cn_g'], blk['cn_b'],
                                 relwc, blk['root_W'])
        parts = _sc_rgcn(tabs.reshape(R * N, C), idx2, dst2, zeros)
        h, st_h = _combine(h, parts, root, blk['conv_bias'].reshape(1, C), m)
        z1, st_z1 = _stage(h, m, blk['m_W0'], blk['m_b0'],
                           norm=(st_h, blk['mn_g'], blk['mn_b']))
        h, st_h = _stage(z1, m, blk['m_W1'], blk['m_b1'],
                         norm=(st_z1, blk['m_g0'], blk['m_be0']), res=h)

    u, st_u = _stage(h, m, p['ro_W0'], p['ro_b0'])
    out, _ = _stage(u, m, p['ro_W1'], p['ro_b1'],
                    norm=(st_u, p['ro_g0'], p['ro_be0']))
    return out


# speedup vs baseline: 10.5357x; 1.3947x over previous
import functools

import jax
import jax.numpy as jnp
from jax import lax
from jax.experimental import pallas as pl
from jax.experimental.pallas import tpu as pltpu
from jax.experimental.pallas import tpu_sc as plsc

N, E, D, T, R, C, H, OUT = 10000, 320000, 128, 3, 4, 32, 256, 32

NB = 5
BN = N // NB
NW = 32
CHUNK = 128
E_PAD = 327680
CPW0 = 104
CPW1 = 56
N_PAD = 10112
RPS = N_PAD // 16

def _dot1(a, b):
    return jax.lax.dot_general(a, b, (((a.ndim - 1,), (0,)), ((), ())),
                               preferred_element_type=jnp.float32)


def _split(a):
    ah = a.astype(jnp.bfloat16)
    al = (a - ah.astype(jnp.float32)).astype(jnp.bfloat16)
    return ah, al


def _stack_w(w):
    wh, wl = _split(w)
    return jnp.concatenate([wh, wh, wl], axis=-2)


def _x3_lhs(x):
    ah, al = _split(x)
    return [ah, al, ah]


def _sel(m, tab):
    out = m[:, 0:1] * tab[0:1, :]
    for t in range(1, T):
        out = out + m[:, t:t + 1] * tab[t:t + 1, :]
    return out


def _leaky(v):
    return jnp.where(v >= 0, v, 0.01 * v)


def _norm_coefs(st, g, be):
    cnt = jnp.maximum(st[2], 1.0)
    mean = st[0] / cnt
    var = jnp.maximum(st[1] / cnt - mean * mean, 0.0)
    sc = jax.lax.rsqrt(var + 1e-5) * g
    return sc, be - mean * sc


def _accum_stats(so_ref, m, y):
    @pl.when(pl.program_id(0) == 0)
    def _():
        so_ref[...] = jnp.zeros_like(so_ref)

    mt = m.T.astype(jnp.bfloat16)
    yb = y.astype(jnp.bfloat16)
    so_ref[0] += _dot1(mt, yb)
    so_ref[1] += _dot1(mt, (y * y).astype(jnp.bfloat16))
    so_ref[2] += _dot1(mt, jnp.ones_like(yb))


def _stage_body(norm, res, x_ref, m_ref, w_ref, b_ref, *rest):
    i = 0
    if norm:
        st_ref, g_ref, be_ref = rest[i:i + 3]
        i += 3
    if res:
        r_ref = rest[i]
        i += 1
    y_ref, so_ref = rest[i:i + 2]

    x = x_ref[...]
    m = m_ref[...]
    if norm:
        sc, sh = _norm_coefs(st_ref[...], g_ref[...], be_ref[...])
        x = _leaky(x * _sel(m, sc) + _sel(m, sh))
    mb = m.astype(jnp.bfloat16)
    halves = _x3_lhs(x)
    pieces = [h * mb[:, t:t + 1] for t in range(T) for h in halves]
    y = _dot1(jnp.concatenate(pieces, axis=1), w_ref[...])
    y += _sel(m, b_ref[...])
    if res:
        y += r_ref[...]
    y_ref[...] = y
    _accum_stats(so_ref, m, y)


def _stage(x, m, w, b, norm=None, res=None):
    n, fin = x.shape
    fout = w.shape[2]
    wstk = _stack_w(w).reshape(T * 3 * fin, fout)
    row = lambda i: (i, 0)
    full3 = pl.BlockSpec(None, lambda i: (0, 0, 0))
    full2 = pl.BlockSpec(None, lambda i: (0, 0))
    in_specs = [pl.BlockSpec((BN, fin), row), pl.BlockSpec((BN, T), row),
                full2, full2]
    args = [x, m, wstk, b]
    if norm is not None:
        st, g, be = norm
        in_specs += [full3, full2, full2]
        args += [st, g, be]
    if res is not None:
        in_specs.append(pl.BlockSpec((BN, fout), row))
        args.append(res)
    return pl.pallas_call(
        functools.partial(_stage_body, norm is not None, res is not None),
        grid=(NB,),
        in_specs=in_specs,
        out_specs=[pl.BlockSpec((BN, fout), row),
                   pl.BlockSpec((3, T, fout), lambda i: (0, 0, 0))],
        out_shape=[jax.ShapeDtypeStruct((n, fout), jnp.float32),
                   jax.ShapeDtypeStruct((3, T, fout), jnp.float32)],
    )(*args)


def _rgcn_dense_body(h_ref, m_ref, st_ref, g_ref, be_ref, w_ref,
                     tab_ref, root_ref):
    m = m_ref[...]
    sc, sh = _norm_coefs(st_ref[...], g_ref[...], be_ref[...])
    y = _leaky(h_ref[...] * _sel(m, sc) + _sel(m, sh))
    out = _dot1(jnp.concatenate(_x3_lhs(y), axis=1), w_ref[...])
    tab_ref[...] = out[:, :R * C]
    root_ref[...] = out[:, R * C:]


def _rgcn_dense(h, m, st, g, be, relwc, rootw):
    wstk = _stack_w(jnp.concatenate([relwc, rootw], axis=1))
    row = lambda i: (i, 0)
    full3 = pl.BlockSpec(None, lambda i: (0, 0, 0))
    full2 = pl.BlockSpec(None, lambda i: (0, 0))
    return pl.pallas_call(
        _rgcn_dense_body,
        grid=(NB,),
        in_specs=[pl.BlockSpec((BN, C), row), pl.BlockSpec((BN, T), row),
                  full3, full2, full2, full2],
        out_specs=[pl.BlockSpec((BN, R * C), row),
                   pl.BlockSpec((BN, C), row)],
        out_shape=[jax.ShapeDtypeStruct((N, R * C), jnp.float32),
                   jax.ShapeDtypeStruct((N, C), jnp.float32)],
    )(h, m, st, g, be, wstk)


def _combine_body(h_ref, p_ref, root_ref, b_ref, m_ref, y_ref, so_ref):
    y = h_ref[...] + p_ref[0] + p_ref[1] + root_ref[...] + b_ref[...]
    y_ref[...] = y
    _accum_stats(so_ref, m_ref[...], y)


def _combine(h, parts, root, bias, m):
    row = lambda i: (i, 0)
    return pl.pallas_call(
        _combine_body,
        grid=(NB,),
        in_specs=[pl.BlockSpec((BN, C), row),
                  pl.BlockSpec((2, BN, C), lambda i: (0, i, 0)),
                  pl.BlockSpec((BN, C), row),
                  pl.BlockSpec(None, lambda i: (0, 0)),
                  pl.BlockSpec((BN, T), row)],
        out_specs=[pl.BlockSpec((BN, C), row),
                   pl.BlockSpec((3, T, C), lambda i: (0, 0, 0))],
        out_shape=[jax.ShapeDtypeStruct((N, C), jnp.float32),
                   jax.ShapeDtypeStruct((3, T, C), jnp.float32)],
    )(h, parts, root, bias, m)


def _prep_body(et_ref, src_ref, nt_ref, idx_ref, m_ref):
    idx_ref[...] = src_ref[...] * R + et_ref[...]
    tt = jax.lax.broadcasted_iota(jnp.int32, (N, T), 1)
    m_ref[...] = (nt_ref[...] == tt).astype(jnp.float32)


def _prep(et2, src2, nt):
    return pl.pallas_call(
        _prep_body,
        out_shape=[jax.ShapeDtypeStruct((E_PAD // CHUNK, CHUNK), jnp.int32),
                   jax.ShapeDtypeStruct((N, T), jnp.float32)],
    )(et2, src2, nt)


def _sc_rgcn_body(tab_ref, idx_ref, dst_ref, zero_ref, out_ref,
                  idx_v, dst_v, rows_v, acc, sem):
    c = lax.axis_index("c")
    s = lax.axis_index("s")

    pltpu.sync_copy(zero_ref.at[pl.ds(s * RPS, RPS)], acc.at[pl.ds(s * RPS, RPS)])

    @pl.when(c == 0)
    def _():
        pltpu.sync_copy(idx_ref.at[pl.ds(s * CPW0, CPW0)],
                        idx_v.at[pl.ds(0, CPW0)])
        pltpu.sync_copy(dst_ref.at[pl.ds(s * CPW0, CPW0)],
                        dst_v.at[pl.ds(0, CPW0)])

    @pl.when(c == 1)
    def _():
        pltpu.sync_copy(idx_ref.at[pl.ds(16 * CPW0 + s * CPW1, CPW1)],
                        idx_v.at[pl.ds(0, CPW1)])
        pltpu.sync_copy(dst_ref.at[pl.ds(16 * CPW0 + s * CPW1, CPW1)],
                        dst_v.at[pl.ds(0, CPW1)])

    plsc.subcore_barrier()
    n = jnp.where(c == 0, CPW0, CPW1)

    def body(j, carry):
        pltpu.async_copy(tab_ref.at[idx_v.at[j]], rows_v, sem).wait()
        pltpu.sync_copy(rows_v, acc.at[dst_v.at[j]], add=True)
        return carry

    lax.fori_loop(0, n, body, 0, unroll=False)
    plsc.subcore_barrier()
    pltpu.sync_copy(acc.at[pl.ds(s * RPS, RPS)],
                    out_ref.at[c].at[pl.ds(s * RPS, RPS)])


@functools.cache
def _sc_rgcn_kernel():
    return pl.kernel(
        _sc_rgcn_body,
        out_type=jax.ShapeDtypeStruct((2, N_PAD, C), jnp.float32),
        mesh=plsc.VectorSubcoreMesh(core_axis_name="c", subcore_axis_name="s"),
        scratch_types=[
            pltpu.VMEM((max(CPW0, CPW1), CHUNK), jnp.int32),
            pltpu.VMEM((max(CPW0, CPW1), CHUNK), jnp.int32),
            pltpu.VMEM((CHUNK, C), jnp.float32),
            pltpu.VMEM_SHARED((N_PAD, C), jnp.float32),
            pltpu.SemaphoreType.DMA,
        ],
        compiler_params=pltpu.CompilerParams(use_tc_tiling_on_sc=False),
    )


def _sc_rgcn(tabs_flat, idx2, dst2, zeros):
    return _sc_rgcn_kernel()(tabs_flat, idx2, dst2, zeros)


def kernel(x, params, edge_index, node_type, edge_type):
    p = params
    f32, i32 = jnp.float32, jnp.int32

    pad = E_PAD - E
    src2 = jnp.concatenate([edge_index[0], jnp.zeros((pad,), i32)]
                           ).reshape(E_PAD // CHUNK, CHUNK)
    dst2 = jnp.concatenate([edge_index[1], jnp.full((pad,), N, i32)]
                           ).reshape(E_PAD // CHUNK, CHUNK)
    et2 = jnp.concatenate([edge_type, jnp.zeros((pad,), i32)]
                          ).reshape(E_PAD // CHUNK, CHUNK)
    zeros = jnp.zeros((N_PAD, C), f32)

    idx2, m = _prep(et2, src2, node_type.reshape(N, 1))

    s0, st0 = _stage(x, m, p['ri_W0'], p['ri_b0'])
    h, st_h = _stage(s0, m, p['ri_W1'], p['ri_b1'],
                     norm=(st0, p['ri_g0'], p['ri_be0']))

    for blk in p['blocks']:
        relwc = blk['rel_W'].transpose(1, 0, 2).reshape(C, R * C)
        tabs, root = _rgcn_dense(h, m, st_h, blk['cn_g'], blk['cn_b'],
                                 relwc, blk['root_W'])
        parts = _sc_rgcn(tabs.reshape(R * N, C), idx2, dst2, zeros)
        h, st_h = _combine(h, parts, root, blk['conv_bias'].reshape(1, C), m)
        z1, st_z1 = _stage(h, m, blk['m_W0'], blk['m_b0'],
                           norm=(st_h, blk['mn_g'], blk['mn_b']))
        h, st_h = _stage(z1, m, blk['m_W1'], blk['m_b1'],
                         norm=(st_z1, blk['m_g0'], blk['m_be0']), res=h)

    u, st_u = _stage(h, m, p['ro_W0'], p['ro_b0'])
    out, _ = _stage(u, m, p['ro_W1'], p['ro_b1'],
                    norm=(st_u, p['ro_g0'], p['ro_be0']))
    return out


# speedup vs baseline: 11.6982x; 1.1103x over previous
import functools

import jax
import jax.numpy as jnp
from jax import lax
from jax.experimental import pallas as pl
from jax.experimental.pallas import tpu as pltpu
from jax.experimental.pallas import tpu_sc as plsc

N, E, D, T, R, C, H, OUT = 10000, 320000, 128, 3, 4, 32, 256, 32

NB = 5
BN = N // NB
NW = 32
CHUNK = 128
E_PAD = 327680
CPW0 = 104
CPW1 = 56
N_PAD = 10112
RPS = N_PAD // 16

def _dot1(a, b):
    return jax.lax.dot_general(a, b, (((a.ndim - 1,), (0,)), ((), ())),
                               preferred_element_type=jnp.float32)


def _split(a):
    ah = a.astype(jnp.bfloat16)
    al = (a - ah.astype(jnp.float32)).astype(jnp.bfloat16)
    return ah, al


def _stack_w(w):
    wh, wl = _split(w)
    return jnp.concatenate([wh, wh, wl], axis=-2)


def _x3_lhs(x):
    ah, al = _split(x)
    return [ah, al, ah]


def _sel(m, tab):
    out = m[:, 0:1] * tab[0:1, :]
    for t in range(1, T):
        out = out + m[:, t:t + 1] * tab[t:t + 1, :]
    return out


def _leaky(v):
    return jnp.where(v >= 0, v, 0.01 * v)


def _norm_coefs(st, g, be):
    cnt = jnp.maximum(st[2], 1.0)
    mean = st[0] / cnt
    var = jnp.maximum(st[1] / cnt - mean * mean, 0.0)
    sc = jax.lax.rsqrt(var + 1e-5) * g
    return sc, be - mean * sc


def _accum_stats(so_ref, m, y):
    @pl.when(pl.program_id(0) == 0)
    def _():
        so_ref[...] = jnp.zeros_like(so_ref)

    mt = m.T.astype(jnp.bfloat16)
    yb = y.astype(jnp.bfloat16)
    so_ref[0] += _dot1(mt, yb)
    so_ref[1] += _dot1(mt, (y * y).astype(jnp.bfloat16))
    so_ref[2] += _dot1(mt, jnp.ones_like(yb))


def _stage_body(norm, res, x_ref, m_ref, w_ref, b_ref, *rest):
    i = 0
    if norm:
        st_ref, g_ref, be_ref = rest[i:i + 3]
        i += 3
    if res:
        r_ref = rest[i]
        i += 1
    y_ref, so_ref = rest[i:i + 2]

    x = x_ref[...]
    m = m_ref[...]
    if norm:
        sc, sh = _norm_coefs(st_ref[...], g_ref[...], be_ref[...])
        x = _leaky(x * _sel(m, sc) + _sel(m, sh))
    mb = m.astype(jnp.bfloat16)
    halves = _x3_lhs(x)
    pieces = [h * mb[:, t:t + 1] for t in range(T) for h in halves]
    y = _dot1(jnp.concatenate(pieces, axis=1), w_ref[...])
    y += _sel(m, b_ref[...])
    if res:
        y += r_ref[...]
    y_ref[...] = y
    _accum_stats(so_ref, m, y)


def _stage(x, m, w, b, norm=None, res=None):
    n, fin = x.shape
    fout = w.shape[2]
    wstk = _stack_w(w).reshape(T * 3 * fin, fout)
    row = lambda i: (i, 0)
    full3 = pl.BlockSpec(None, lambda i: (0, 0, 0))
    full2 = pl.BlockSpec(None, lambda i: (0, 0))
    in_specs = [pl.BlockSpec((BN, fin), row), pl.BlockSpec((BN, T), row),
                full2, full2]
    args = [x, m, wstk, b]
    if norm is not None:
        st, g, be = norm
        in_specs += [full3, full2, full2]
        args += [st, g, be]
    if res is not None:
        in_specs.append(pl.BlockSpec((BN, fout), row))
        args.append(res)
    return pl.pallas_call(
        functools.partial(_stage_body, norm is not None, res is not None),
        grid=(NB,),
        in_specs=in_specs,
        out_specs=[pl.BlockSpec((BN, fout), row),
                   pl.BlockSpec((3, T, fout), lambda i: (0, 0, 0))],
        out_shape=[jax.ShapeDtypeStruct((n, fout), jnp.float32),
                   jax.ShapeDtypeStruct((3, T, fout), jnp.float32)],
    )(*args)


def _rgcn_dense_body(h_ref, m_ref, st_ref, g_ref, be_ref, w_ref,
                     tab_ref, root_ref):
    m = m_ref[...]
    sc, sh = _norm_coefs(st_ref[...], g_ref[...], be_ref[...])
    y = _leaky(h_ref[...] * _sel(m, sc) + _sel(m, sh))
    out = _dot1(jnp.concatenate(_x3_lhs(y), axis=1), w_ref[...])
    tab_ref[...] = out[:, :R * C]
    root_ref[...] = out[:, R * C:]


def _rgcn_dense(h, m, st, g, be, relwc, rootw):
    wstk = _stack_w(jnp.concatenate([relwc, rootw], axis=1))
    row = lambda i: (i, 0)
    full3 = pl.BlockSpec(None, lambda i: (0, 0, 0))
    full2 = pl.BlockSpec(None, lambda i: (0, 0))
    return pl.pallas_call(
        _rgcn_dense_body,
        grid=(NB,),
        in_specs=[pl.BlockSpec((BN, C), row), pl.BlockSpec((BN, T), row),
                  full3, full2, full2, full2],
        out_specs=[pl.BlockSpec((BN, R * C), row),
                   pl.BlockSpec((BN, C), row)],
        out_shape=[jax.ShapeDtypeStruct((N, R * C), jnp.float32),
                   jax.ShapeDtypeStruct((N, C), jnp.float32)],
    )(h, m, st, g, be, wstk)


def _combine_body(h_ref, p_ref, root_ref, b_ref, m_ref, y_ref, so_ref):
    y = h_ref[...] + p_ref[0] + p_ref[1] + root_ref[...] + b_ref[...]
    y_ref[...] = y
    _accum_stats(so_ref, m_ref[...], y)


def _combine(h, parts, root, bias, m):
    row = lambda i: (i, 0)
    return pl.pallas_call(
        _combine_body,
        grid=(NB,),
        in_specs=[pl.BlockSpec((BN, C), row),
                  pl.BlockSpec((2, BN, C), lambda i: (0, i, 0)),
                  pl.BlockSpec((BN, C), row),
                  pl.BlockSpec(None, lambda i: (0, 0)),
                  pl.BlockSpec((BN, T), row)],
        out_specs=[pl.BlockSpec((BN, C), row),
                   pl.BlockSpec((3, T, C), lambda i: (0, 0, 0))],
        out_shape=[jax.ShapeDtypeStruct((N, C), jnp.float32),
                   jax.ShapeDtypeStruct((3, T, C), jnp.float32)],
    )(h, parts, root, bias, m)


def _prep_body(et_ref, src_ref, nt_ref, idx_ref, m_ref):
    idx_ref[...] = src_ref[...] * R + et_ref[...]
    tt = jax.lax.broadcasted_iota(jnp.int32, (N, T), 1)
    m_ref[...] = (nt_ref[...] == tt).astype(jnp.float32)


def _prep(et2, src2, nt):
    return pl.pallas_call(
        _prep_body,
        out_shape=[jax.ShapeDtypeStruct((E_PAD // CHUNK, CHUNK), jnp.int32),
                   jax.ShapeDtypeStruct((N, T), jnp.float32)],
    )(et2, src2, nt)


def _sc_rgcn_body(tab_ref, idx_ref, dst_ref, zero_ref, out_ref,
                  idx_v, dst_v, rows_v, acc, sem0, sem1):
    c = lax.axis_index("c")
    s = lax.axis_index("s")

    pltpu.sync_copy(zero_ref.at[pl.ds(s * RPS, RPS)], acc.at[pl.ds(s * RPS, RPS)])

    @pl.when(c == 0)
    def _():
        pltpu.sync_copy(idx_ref.at[pl.ds(s * CPW0, CPW0)],
                        idx_v.at[pl.ds(0, CPW0)])
        pltpu.sync_copy(dst_ref.at[pl.ds(s * CPW0, CPW0)],
                        dst_v.at[pl.ds(0, CPW0)])

    @pl.when(c == 1)
    def _():
        pltpu.sync_copy(idx_ref.at[pl.ds(16 * CPW0 + s * CPW1, CPW1)],
                        idx_v.at[pl.ds(0, CPW1)])
        pltpu.sync_copy(dst_ref.at[pl.ds(16 * CPW0 + s * CPW1, CPW1)],
                        dst_v.at[pl.ds(0, CPW1)])

    plsc.subcore_barrier()
    n = jnp.where(c == 0, CPW0, CPW1)

    pltpu.async_copy(tab_ref.at[idx_v.at[0]], rows_v.at[0], sem0)

    def body(i, carry):
        j = 2 * i

        @pl.when(j + 1 < n)
        def _():
            pltpu.async_copy(tab_ref.at[idx_v.at[j + 1]], rows_v.at[1], sem1)

        pltpu.make_async_copy(tab_ref.at[idx_v.at[j]], rows_v.at[0],
                              sem0).wait()
        pltpu.sync_copy(rows_v.at[0], acc.at[dst_v.at[j]], add=True)

        @pl.when(j + 2 < n)
        def _():
            pltpu.async_copy(tab_ref.at[idx_v.at[j + 2]], rows_v.at[0], sem0)

        @pl.when(j + 1 < n)
        def _():
            pltpu.make_async_copy(tab_ref.at[idx_v.at[j + 1]], rows_v.at[1],
                                  sem1).wait()
            pltpu.sync_copy(rows_v.at[1], acc.at[dst_v.at[j + 1]], add=True)

        return carry

    lax.fori_loop(0, (n + 1) // 2, body, 0, unroll=False)
    plsc.subcore_barrier()
    pltpu.sync_copy(acc.at[pl.ds(s * RPS, RPS)],
                    out_ref.at[c].at[pl.ds(s * RPS, RPS)])


@functools.cache
def _sc_rgcn_kernel():
    return pl.kernel(
        _sc_rgcn_body,
        out_type=jax.ShapeDtypeStruct((2, N_PAD, C), jnp.float32),
        mesh=plsc.VectorSubcoreMesh(core_axis_name="c", subcore_axis_name="s"),
        scratch_types=[
            pltpu.VMEM((max(CPW0, CPW1), CHUNK), jnp.int32),
            pltpu.VMEM((max(CPW0, CPW1), CHUNK), jnp.int32),
            pltpu.VMEM((2, CHUNK, C), jnp.float32),
            pltpu.VMEM_SHARED((N_PAD, C), jnp.float32),
            pltpu.SemaphoreType.DMA,
            pltpu.SemaphoreType.DMA,
        ],
        compiler_params=pltpu.CompilerParams(use_tc_tiling_on_sc=False),
    )


def _sc_rgcn(tabs_flat, idx2, dst2, zeros):
    return _sc_rgcn_kernel()(tabs_flat, idx2, dst2, zeros)


def kernel(x, params, edge_index, node_type, edge_type):
    p = params
    f32, i32 = jnp.float32, jnp.int32

    pad = E_PAD - E
    src2 = jnp.concatenate([edge_index[0], jnp.zeros((pad,), i32)]
                           ).reshape(E_PAD // CHUNK, CHUNK)
    dst2 = jnp.concatenate([edge_index[1], jnp.full((pad,), N, i32)]
                           ).reshape(E_PAD // CHUNK, CHUNK)
    et2 = jnp.concatenate([edge_type, jnp.zeros((pad,), i32)]
                          ).reshape(E_PAD // CHUNK, CHUNK)
    zeros = jnp.zeros((N_PAD, C), f32)

    idx2, m = _prep(et2, src2, node_type.reshape(N, 1))

    s0, st0 = _stage(x, m, p['ri_W0'], p['ri_b0'])
    h, st_h = _stage(s0, m, p['ri_W1'], p['ri_b1'],
                     norm=(st0, p['ri_g0'], p['ri_be0']))

    for blk in p['blocks']:
        relwc = blk['rel_W'].transpose(1, 0, 2).reshape(C, R * C)
        tabs, root = _rgcn_dense(h, m, st_h, blk['cn_g'], blk['cn_b'],
                                 relwc, blk['root_W'])
        parts = _sc_rgcn(tabs.reshape(R * N, C), idx2, dst2, zeros)
        h, st_h = _combine(h, parts, root, blk['conv_bias'].reshape(1, C), m)
        z1, st_z1 = _stage(h, m, blk['m_W0'], blk['m_b0'],
                           norm=(st_h, blk['mn_g'], blk['mn_b']))
        h, st_h = _stage(z1, m, blk['m_W1'], blk['m_b1'],
                         norm=(st_z1, blk['m_g0'], blk['m_be0']), res=h)

    u, st_u = _stage(h, m, p['ro_W0'], p['ro_b0'])
    out, _ = _stage(u, m, p['ro_W1'], p['ro_b1'],
                    norm=(st_u, p['ro_g0'], p['ro_be0']))
    return out


# speedup vs baseline: 11.8686x; 1.0146x over previous
import functools

import jax
import jax.numpy as jnp
from jax import lax
from jax.experimental import pallas as pl
from jax.experimental.pallas import tpu as pltpu
from jax.experimental.pallas import tpu_sc as plsc

N, E, D, T, R, C, H, OUT = 10000, 320000, 128, 3, 4, 32, 256, 32

NB = 5
BN = N // NB
NW = 32
CHUNK = 128
E_PAD = 327680
CPW0 = 112
CPW1 = 48
N_PAD = 10112
RPS = N_PAD // 16

def _dot1(a, b):
    return jax.lax.dot_general(a, b, (((a.ndim - 1,), (0,)), ((), ())),
                               preferred_element_type=jnp.float32)


def _split(a):
    ah = a.astype(jnp.bfloat16)
    al = (a - ah.astype(jnp.float32)).astype(jnp.bfloat16)
    return ah, al


def _stack_w(w):
    wh, wl = _split(w)
    return jnp.concatenate([wh, wh, wl], axis=-2)


def _x3_lhs(x):
    ah, al = _split(x)
    return [ah, al, ah]


def _sel(m, tab):
    out = m[:, 0:1] * tab[0:1, :]
    for t in range(1, T):
        out = out + m[:, t:t + 1] * tab[t:t + 1, :]
    return out


def _sel_many(m2, tabs):
    hi, lo = _split(jnp.concatenate(tabs, axis=1))
    out = _dot1(m2, jnp.concatenate([hi, lo], axis=0))
    res, off = [], 0
    for tab in tabs:
        res.append(out[:, off:off + tab.shape[1]])
        off += tab.shape[1]
    return res


def _leaky(v):
    return jnp.where(v >= 0, v, 0.01 * v)


def _norm_coefs(st, g, be):
    cnt = jnp.maximum(st[2], 1.0)
    mean = st[0] / cnt
    var = jnp.maximum(st[1] / cnt - mean * mean, 0.0)
    sc = jax.lax.rsqrt(var + 1e-5) * g
    return sc, be - mean * sc


def _accum_stats(so_ref, m, y):
    @pl.when(pl.program_id(0) == 0)
    def _():
        so_ref[...] = jnp.zeros_like(so_ref)

    mt = m.T.astype(jnp.bfloat16)
    yb = y.astype(jnp.bfloat16)
    so_ref[0] += _dot1(mt, yb)
    so_ref[1] += _dot1(mt, (y * y).astype(jnp.bfloat16))
    so_ref[2] += _dot1(mt, jnp.ones_like(yb))


def _stage_body(norm, res, x_ref, m_ref, w_ref, b_ref, *rest):
    i = 0
    if norm:
        st_ref, g_ref, be_ref = rest[i:i + 3]
        i += 3
    if res:
        r_ref = rest[i]
        i += 1
    y_ref, so_ref = rest[i:i + 2]

    x = x_ref[...]
    m = m_ref[...]
    mb = m.astype(jnp.bfloat16)
    m2 = jnp.concatenate([mb, mb], axis=1)
    if norm:
        sc, sh = _norm_coefs(st_ref[...], g_ref[...], be_ref[...])
        a, bsh, bias = _sel_many(m2, [sc, sh, b_ref[...]])
        x = _leaky(x * a + bsh)
    else:
        bias = _sel_many(m2, [b_ref[...]])[0]
    halves = _x3_lhs(x)
    pieces = [h * mb[:, t:t + 1] for t in range(T) for h in halves]
    y = _dot1(jnp.concatenate(pieces, axis=1), w_ref[...])
    y += bias
    if res:
        y += r_ref[...]
    y_ref[...] = y
    _accum_stats(so_ref, m, y)


def _stage(x, m, w, b, norm=None, res=None):
    n, fin = x.shape
    fout = w.shape[2]
    wstk = _stack_w(w).reshape(T * 3 * fin, fout)
    row = lambda i: (i, 0)
    full3 = pl.BlockSpec(None, lambda i: (0, 0, 0))
    full2 = pl.BlockSpec(None, lambda i: (0, 0))
    in_specs = [pl.BlockSpec((BN, fin), row), pl.BlockSpec((BN, T), row),
                full2, full2]
    args = [x, m, wstk, b]
    if norm is not None:
        st, g, be = norm
        in_specs += [full3, full2, full2]
        args += [st, g, be]
    if res is not None:
        in_specs.append(pl.BlockSpec((BN, fout), row))
        args.append(res)
    return pl.pallas_call(
        functools.partial(_stage_body, norm is not None, res is not None),
        grid=(NB,),
        in_specs=in_specs,
        out_specs=[pl.BlockSpec((BN, fout), row),
                   pl.BlockSpec((3, T, fout), lambda i: (0, 0, 0))],
        out_shape=[jax.ShapeDtypeStruct((n, fout), jnp.float32),
                   jax.ShapeDtypeStruct((3, T, fout), jnp.float32)],
    )(*args)


def _rgcn_dense_body(h_ref, m_ref, st_ref, g_ref, be_ref, w_ref,
                     tab_ref, root_ref):
    m = m_ref[...]
    mb = m.astype(jnp.bfloat16)
    sc, sh = _norm_coefs(st_ref[...], g_ref[...], be_ref[...])
    a, bsh = _sel_many(jnp.concatenate([mb, mb], axis=1), [sc, sh])
    y = _leaky(h_ref[...] * a + bsh)
    out = _dot1(jnp.concatenate(_x3_lhs(y), axis=1), w_ref[...])
    tab_ref[...] = out[:, :R * C]
    root_ref[...] = out[:, R * C:]


def _rgcn_dense(h, m, st, g, be, relwc, rootw):
    wstk = _stack_w(jnp.concatenate([relwc, rootw], axis=1))
    row = lambda i: (i, 0)
    full3 = pl.BlockSpec(None, lambda i: (0, 0, 0))
    full2 = pl.BlockSpec(None, lambda i: (0, 0))
    return pl.pallas_call(
        _rgcn_dense_body,
        grid=(NB,),
        in_specs=[pl.BlockSpec((BN, C), row), pl.BlockSpec((BN, T), row),
                  full3, full2, full2, full2],
        out_specs=[pl.BlockSpec((BN, R * C), row),
                   pl.BlockSpec((BN, C), row)],
        out_shape=[jax.ShapeDtypeStruct((N, R * C), jnp.float32),
                   jax.ShapeDtypeStruct((N, C), jnp.float32)],
    )(h, m, st, g, be, wstk)


def _combine_body(h_ref, p_ref, root_ref, b_ref, m_ref, y_ref, so_ref):
    y = h_ref[...] + p_ref[0] + p_ref[1] + root_ref[...] + b_ref[...]
    y_ref[...] = y
    _accum_stats(so_ref, m_ref[...], y)


def _combine(h, parts, root, bias, m):
    row = lambda i: (i, 0)
    return pl.pallas_call(
        _combine_body,
        grid=(NB,),
        in_specs=[pl.BlockSpec((BN, C), row),
                  pl.BlockSpec((2, BN, C), lambda i: (0, i, 0)),
                  pl.BlockSpec((BN, C), row),
                  pl.BlockSpec(None, lambda i: (0, 0)),
                  pl.BlockSpec((BN, T), row)],
        out_specs=[pl.BlockSpec((BN, C), row),
                   pl.BlockSpec((3, T, C), lambda i: (0, 0, 0))],
        out_shape=[jax.ShapeDtypeStruct((N, C), jnp.float32),
                   jax.ShapeDtypeStruct((3, T, C), jnp.float32)],
    )(h, parts, root, bias, m)


def _prep_body(et_ref, src_ref, nt_ref, idx_ref, m_ref):
    idx_ref[...] = src_ref[...] * R + et_ref[...]
    tt = jax.lax.broadcasted_iota(jnp.int32, (N, T), 1)
    m_ref[...] = (nt_ref[...] == tt).astype(jnp.float32)


def _prep(et2, src2, nt):
    return pl.pallas_call(
        _prep_body,
        out_shape=[jax.ShapeDtypeStruct((E_PAD // CHUNK, CHUNK), jnp.int32),
                   jax.ShapeDtypeStruct((N, T), jnp.float32)],
    )(et2, src2, nt)


def _sc_rgcn_body(tab_ref, idx_ref, dst_ref, zero_ref, out_ref,
                  idx_v, dst_v, rows_v, acc, sem0, sem1):
    c = lax.axis_index("c")
    s = lax.axis_index("s")

    pltpu.sync_copy(zero_ref.at[pl.ds(s * RPS, RPS)], acc.at[pl.ds(s * RPS, RPS)])

    @pl.when(c == 0)
    def _():
        pltpu.sync_copy(idx_ref.at[pl.ds(s * CPW0, CPW0)],
                        idx_v.at[pl.ds(0, CPW0)])
        pltpu.sync_copy(dst_ref.at[pl.ds(s * CPW0, CPW0)],
                        dst_v.at[pl.ds(0, CPW0)])

    @pl.when(c == 1)
    def _():
        pltpu.sync_copy(idx_ref.at[pl.ds(16 * CPW0 + s * CPW1, CPW1)],
                        idx_v.at[pl.ds(0, CPW1)])
        pltpu.sync_copy(dst_ref.at[pl.ds(16 * CPW0 + s * CPW1, CPW1)],
                        dst_v.at[pl.ds(0, CPW1)])

    plsc.subcore_barrier()
    n = jnp.where(c == 0, CPW0, CPW1)

    pltpu.async_copy(tab_ref.at[idx_v.at[0]], rows_v.at[0], sem0)

    def body(i, carry):
        j = 2 * i

        @pl.when(j + 1 < n)
        def _():
            pltpu.async_copy(tab_ref.at[idx_v.at[j + 1]], rows_v.at[1], sem1)

        pltpu.make_async_copy(tab_ref.at[idx_v.at[j]], rows_v.at[0],
                              sem0).wait()
        pltpu.sync_copy(rows_v.at[0], acc.at[dst_v.at[j]], add=True)

        @pl.when(j + 2 < n)
        def _():
            pltpu.async_copy(tab_ref.at[idx_v.at[j + 2]], rows_v.at[0], sem0)

        @pl.when(j + 1 < n)
        def _():
            pltpu.make_async_copy(tab_ref.at[idx_v.at[j + 1]], rows_v.at[1],
                                  sem1).wait()
            pltpu.sync_copy(rows_v.at[1], acc.at[dst_v.at[j + 1]], add=True)

        return carry

    lax.fori_loop(0, (n + 1) // 2, body, 0, unroll=False)
    plsc.subcore_barrier()
    pltpu.sync_copy(acc.at[pl.ds(s * RPS, RPS)],
                    out_ref.at[c].at[pl.ds(s * RPS, RPS)])


@functools.cache
def _sc_rgcn_kernel():
    return pl.kernel(
        _sc_rgcn_body,
        out_type=jax.ShapeDtypeStruct((2, N_PAD, C), jnp.float32),
        mesh=plsc.VectorSubcoreMesh(core_axis_name="c", subcore_axis_name="s"),
        scratch_types=[
            pltpu.VMEM((max(CPW0, CPW1), CHUNK), jnp.int32),
            pltpu.VMEM((max(CPW0, CPW1), CHUNK), jnp.int32),
            pltpu.VMEM((2, CHUNK, C), jnp.float32),
            pltpu.VMEM_SHARED((N_PAD, C), jnp.float32),
            pltpu.SemaphoreType.DMA,
            pltpu.SemaphoreType.DMA,
        ],
        compiler_params=pltpu.CompilerParams(use_tc_tiling_on_sc=False),
    )


def _sc_rgcn(tabs_flat, idx2, dst2, zeros):
    return _sc_rgcn_kernel()(tabs_flat, idx2, dst2, zeros)


def kernel(x, params, edge_index, node_type, edge_type):
    p = params
    f32, i32 = jnp.float32, jnp.int32

    pad = E_PAD - E
    src2 = jnp.concatenate([edge_index[0], jnp.zeros((pad,), i32)]
                           ).reshape(E_PAD // CHUNK, CHUNK)
    dst2 = jnp.concatenate([edge_index[1], jnp.full((pad,), N, i32)]
                           ).reshape(E_PAD // CHUNK, CHUNK)
    et2 = jnp.concatenate([edge_type, jnp.zeros((pad,), i32)]
                          ).reshape(E_PAD // CHUNK, CHUNK)
    zeros = jnp.zeros((N_PAD, C), f32)

    idx2, m = _prep(et2, src2, node_type.reshape(N, 1))

    s0, st0 = _stage(x, m, p['ri_W0'], p['ri_b0'])
    h, st_h = _stage(s0, m, p['ri_W1'], p['ri_b1'],
                     norm=(st0, p['ri_g0'], p['ri_be0']))

    for blk in p['blocks']:
        relwc = blk['rel_W'].transpose(1, 0, 2).reshape(C, R * C)
        tabs, root = _rgcn_dense(h, m, st_h, blk['cn_g'], blk['cn_b'],
                                 relwc, blk['root_W'])
        parts = _sc_rgcn(tabs.reshape(R * N, C), idx2, dst2, zeros)
        h, st_h = _combine(h, parts, root, blk['conv_bias'].reshape(1, C), m)
        z1, st_z1 = _stage(h, m, blk['m_W0'], blk['m_b0'],
                           norm=(st_h, blk['mn_g'], blk['mn_b']))
        h, st_h = _stage(z1, m, blk['m_W1'], blk['m_b1'],
                         norm=(st_z1, blk['m_g0'], blk['m_be0']), res=h)

    u, st_u = _stage(h, m, p['ro_W0'], p['ro_b0'])
    out, _ = _stage(u, m, p['ro_W1'], p['ro_b1'],
                    norm=(st_u, p['ro_g0'], p['ro_be0']))
    return out


# speedup vs baseline: 11.8715x; 1.0003x over previous
import functools

import jax
import jax.numpy as jnp
from jax import lax
from jax.experimental import pallas as pl
from jax.experimental.pallas import tpu as pltpu
from jax.experimental.pallas import tpu_sc as plsc

N, E, D, T, R, C, H, OUT = 10000, 320000, 128, 3, 4, 32, 256, 32

NB = 5
BN = N // NB
NW = 32
CHUNK = 128
E_PAD = 327680
CPW0 = 112
CPW1 = 48
N_PAD = 10112
RPS = N_PAD // 16

def _dot1(a, b):
    return jax.lax.dot_general(a, b, (((a.ndim - 1,), (0,)), ((), ())),
                               preferred_element_type=jnp.float32)


def _split(a):
    ah = a.astype(jnp.bfloat16)
    al = (a - ah.astype(jnp.float32)).astype(jnp.bfloat16)
    return ah, al


def _stack_w(w):
    wh, wl = _split(w)
    return jnp.concatenate([wh, wh, wl], axis=-2)


def _x3_lhs(x):
    ah, al = _split(x)
    return [ah, al, ah]


def _sel_many(m2, tabs):
    hi, lo = _split(jnp.concatenate(tabs, axis=1))
    out = _dot1(m2, jnp.concatenate([hi, lo], axis=0))
    res, off = [], 0
    for tab in tabs:
        res.append(out[:, off:off + tab.shape[1]])
        off += tab.shape[1]
    return res


def _leaky(v):
    return jnp.where(v >= 0, v, 0.01 * v)


def _norm_coefs(st, g, be):
    cnt = jnp.maximum(st[2], 1.0)
    mean = st[0] / cnt
    var = jnp.maximum(st[1] / cnt - mean * mean, 0.0)
    sc = jax.lax.rsqrt(var + 1e-5) * g
    return sc, be - mean * sc


def _accum_stats(so_ref, m, y):
    @pl.when(pl.program_id(0) == 0)
    def _():
        so_ref[...] = jnp.zeros_like(so_ref)

    mt = m.T.astype(jnp.bfloat16)
    yb = y.astype(jnp.bfloat16)
    so_ref[0] += _dot1(mt, yb)
    so_ref[1] += _dot1(mt, (y * y).astype(jnp.bfloat16))
    so_ref[2] += _dot1(mt, jnp.ones_like(yb))


def _stage_body(norm, res, x_ref, m_ref, w_ref, b_ref, *rest):
    i = 0
    if norm:
        st_ref, g_ref, be_ref = rest[i:i + 3]
        i += 3
    if res:
        r_ref = rest[i]
        i += 1
    y_ref, so_ref = rest[i:i + 2]

    x = x_ref[...]
    m = m_ref[...]
    mb = m.astype(jnp.bfloat16)
    m2 = jnp.concatenate([mb, mb], axis=1)
    if norm:
        sc, sh = _norm_coefs(st_ref[...], g_ref[...], be_ref[...])
        a, bsh, bias = _sel_many(m2, [sc, sh, b_ref[...]])
        x = _leaky(x * a + bsh)
    else:
        bias = _sel_many(m2, [b_ref[...]])[0]
    halves = _x3_lhs(x)
    pieces = [h * mb[:, t:t + 1] for t in range(T) for h in halves]
    y = _dot1(jnp.concatenate(pieces, axis=1), w_ref[...])
    y += bias
    if res:
        y += r_ref[...]
    y_ref[...] = y
    _accum_stats(so_ref, m, y)


def _stage(x, m, w, b, norm=None, res=None):
    n, fin = x.shape
    fout = w.shape[2]
    wstk = _stack_w(w).reshape(T * 3 * fin, fout)
    row = lambda i: (i, 0)
    full3 = pl.BlockSpec(None, lambda i: (0, 0, 0))
    full2 = pl.BlockSpec(None, lambda i: (0, 0))
    in_specs = [pl.BlockSpec((BN, fin), row), pl.BlockSpec((BN, T), row),
                full2, full2]
    args = [x, m, wstk, b]
    if norm is not None:
        st, g, be = norm
        in_specs += [full3, full2, full2]
        args += [st, g, be]
    if res is not None:
        in_specs.append(pl.BlockSpec((BN, fout), row))
        args.append(res)
    return pl.pallas_call(
        functools.partial(_stage_body, norm is not None, res is not None),
        grid=(NB,),
        in_specs=in_specs,
        out_specs=[pl.BlockSpec((BN, fout), row),
                   pl.BlockSpec((3, T, fout), lambda i: (0, 0, 0))],
        out_shape=[jax.ShapeDtypeStruct((n, fout), jnp.float32),
                   jax.ShapeDtypeStruct((3, T, fout), jnp.float32)],
    )(*args)


def _rgcn_dense_body(h_ref, m_ref, st_ref, g_ref, be_ref, w_ref,
                     tab_ref, root_ref):
    m = m_ref[...]
    mb = m.astype(jnp.bfloat16)
    sc, sh = _norm_coefs(st_ref[...], g_ref[...], be_ref[...])
    a, bsh = _sel_many(jnp.concatenate([mb, mb], axis=1), [sc, sh])
    y = _leaky(h_ref[...] * a + bsh)
    out = _dot1(jnp.concatenate(_x3_lhs(y), axis=1), w_ref[...])
    tab_ref[...] = out[:, :R * C]
    root_ref[...] = out[:, R * C:]


def _rgcn_dense(h, m, st, g, be, relwc, rootw):
    wstk = _stack_w(jnp.concatenate([relwc, rootw], axis=1))
    row = lambda i: (i, 0)
    full3 = pl.BlockSpec(None, lambda i: (0, 0, 0))
    full2 = pl.BlockSpec(None, lambda i: (0, 0))
    return pl.pallas_call(
        _rgcn_dense_body,
        grid=(NB,),
        in_specs=[pl.BlockSpec((BN, C), row), pl.BlockSpec((BN, T), row),
                  full3, full2, full2, full2],
        out_specs=[pl.BlockSpec((BN, R * C), row),
                   pl.BlockSpec((BN, C), row)],
        out_shape=[jax.ShapeDtypeStruct((N, R * C), jnp.float32),
                   jax.ShapeDtypeStruct((N, C), jnp.float32)],
    )(h, m, st, g, be, wstk)


def _combine_body(h_ref, p_ref, root_ref, b_ref, m_ref, y_ref, so_ref):
    y = h_ref[...] + p_ref[0] + p_ref[1] + root_ref[...] + b_ref[...]
    y_ref[...] = y
    _accum_stats(so_ref, m_ref[...], y)


def _combine(h, parts, root, bias, m):
    row = lambda i: (i, 0)
    return pl.pallas_call(
        _combine_body,
        grid=(NB,),
        in_specs=[pl.BlockSpec((BN, C), row),
                  pl.BlockSpec((2, BN, C), lambda i: (0, i, 0)),
                  pl.BlockSpec((BN, C), row),
                  pl.BlockSpec(None, lambda i: (0, 0)),
                  pl.BlockSpec((BN, T), row)],
        out_specs=[pl.BlockSpec((BN, C), row),
                   pl.BlockSpec((3, T, C), lambda i: (0, 0, 0))],
        out_shape=[jax.ShapeDtypeStruct((N, C), jnp.float32),
                   jax.ShapeDtypeStruct((3, T, C), jnp.float32)],
    )(h, parts, root, bias, m)


def _prep_body(et_ref, src_ref, nt_ref, idx_ref, m_ref):
    idx_ref[...] = src_ref[...] * R + et_ref[...]
    tt = jax.lax.broadcasted_iota(jnp.int32, (N, T), 1)
    m_ref[...] = (nt_ref[...] == tt).astype(jnp.float32)


def _prep(et2, src2, nt):
    return pl.pallas_call(
        _prep_body,
        out_shape=[jax.ShapeDtypeStruct((E_PAD // CHUNK, CHUNK), jnp.int32),
                   jax.ShapeDtypeStruct((N, T), jnp.float32)],
    )(et2, src2, nt)


def _sc_rgcn_body(tab_ref, idx_ref, dst_ref, zero_ref, out_ref,
                  idx_v, dst_v, rows_v, acc, sem0, sem1):
    c = lax.axis_index("c")
    s = lax.axis_index("s")

    pltpu.sync_copy(zero_ref.at[pl.ds(s * RPS, RPS)], acc.at[pl.ds(s * RPS, RPS)])

    @pl.when(c == 0)
    def _():
        pltpu.sync_copy(idx_ref.at[pl.ds(s * CPW0, CPW0)],
                        idx_v.at[pl.ds(0, CPW0)])
        pltpu.sync_copy(dst_ref.at[pl.ds(s * CPW0, CPW0)],
                        dst_v.at[pl.ds(0, CPW0)])

    @pl.when(c == 1)
    def _():
        pltpu.sync_copy(idx_ref.at[pl.ds(16 * CPW0 + s * CPW1, CPW1)],
                        idx_v.at[pl.ds(0, CPW1)])
        pltpu.sync_copy(dst_ref.at[pl.ds(16 * CPW0 + s * CPW1, CPW1)],
                        dst_v.at[pl.ds(0, CPW1)])

    plsc.subcore_barrier()
    n = jnp.where(c == 0, CPW0, CPW1)

    pltpu.async_copy(tab_ref.at[idx_v.at[0]], rows_v.at[0], sem0)

    def body(i, carry):
        j = 2 * i

        @pl.when(j + 1 < n)
        def _():
            pltpu.async_copy(tab_ref.at[idx_v.at[j + 1]], rows_v.at[1], sem1)

        pltpu.make_async_copy(tab_ref.at[idx_v.at[j]], rows_v.at[0],
                              sem0).wait()
        pltpu.sync_copy(rows_v.at[0], acc.at[dst_v.at[j]], add=True)

        @pl.when(j + 2 < n)
        def _():
            pltpu.async_copy(tab_ref.at[idx_v.at[j + 2]], rows_v.at[0], sem0)

        @pl.when(j + 1 < n)
        def _():
            pltpu.make_async_copy(tab_ref.at[idx_v.at[j + 1]], rows_v.at[1],
                                  sem1).wait()
            pltpu.sync_copy(rows_v.at[1], acc.at[dst_v.at[j + 1]], add=True)

        return carry

    lax.fori_loop(0, (n + 1) // 2, body, 0, unroll=False)
    plsc.subcore_barrier()
    pltpu.sync_copy(acc.at[pl.ds(s * RPS, RPS)],
                    out_ref.at[c].at[pl.ds(s * RPS, RPS)])


@functools.cache
def _sc_rgcn_kernel():
    return pl.kernel(
        _sc_rgcn_body,
        out_type=jax.ShapeDtypeStruct((2, N_PAD, C), jnp.float32),
        mesh=plsc.VectorSubcoreMesh(core_axis_name="c", subcore_axis_name="s"),
        scratch_types=[
            pltpu.VMEM((max(CPW0, CPW1), CHUNK), jnp.int32),
            pltpu.VMEM((max(CPW0, CPW1), CHUNK), jnp.int32),
            pltpu.VMEM((2, CHUNK, C), jnp.float32),
            pltpu.VMEM_SHARED((N_PAD, C), jnp.float32),
            pltpu.SemaphoreType.DMA,
            pltpu.SemaphoreType.DMA,
        ],
        compiler_params=pltpu.CompilerParams(use_tc_tiling_on_sc=False),
    )


def _sc_rgcn(tabs_flat, idx2, dst2, zeros):
    return _sc_rgcn_kernel()(tabs_flat, idx2, dst2, zeros)


def kernel(x, params, edge_index, node_type, edge_type):
    p = params
    f32, i32 = jnp.float32, jnp.int32

    pad = E_PAD - E
    src2 = jnp.concatenate([edge_index[0], jnp.zeros((pad,), i32)]
                           ).reshape(E_PAD // CHUNK, CHUNK)
    dst2 = jnp.concatenate([edge_index[1], jnp.full((pad,), N, i32)]
                           ).reshape(E_PAD // CHUNK, CHUNK)
    et2 = jnp.concatenate([edge_type, jnp.zeros((pad,), i32)]
                          ).reshape(E_PAD // CHUNK, CHUNK)
    zeros = jnp.zeros((N_PAD, C), f32)

    idx2, m = _prep(et2, src2, node_type.reshape(N, 1))

    s0, st0 = _stage(x, m, p['ri_W0'], p['ri_b0'])
    h, st_h = _stage(s0, m, p['ri_W1'], p['ri_b1'],
                     norm=(st0, p['ri_g0'], p['ri_be0']))

    for blk in p['blocks']:
        relwc = blk['rel_W'].transpose(1, 0, 2).reshape(C, R * C)
        tabs, root = _rgcn_dense(h, m, st_h, blk['cn_g'], blk['cn_b'],
                                 relwc, blk['root_W'])
        parts = _sc_rgcn(tabs.reshape(R * N, C), idx2, dst2, zeros)
        h, st_h = _combine(h, parts, root, blk['conv_bias'].reshape(1, C), m)
        z1, st_z1 = _stage(h, m, blk['m_W0'], blk['m_b0'],
                           norm=(st_h, blk['mn_g'], blk['mn_b']))
        h, st_h = _stage(z1, m, blk['m_W1'], blk['m_b1'],
                         norm=(st_z1, blk['m_g0'], blk['m_be0']), res=h)

    u, st_u = _stage(h, m, p['ro_W0'], p['ro_b0'])
    out, _ = _stage(u, m, p['ro_W1'], p['ro_b1'],
                    norm=(st_u, p['ro_g0'], p['ro_be0']))
    return out
